# Initial kernel scaffold; baseline (speedup 1.0000x reference)
import jax
import jax.numpy as jnp
from jax.experimental import pallas as pl


def kernel(x, edge_index, edge_attr, prb_data, unit_normal_vector, tags, params):
    raise NotImplementedError("write your pallas kernel here")


# speedup vs baseline: 1.4309x; 1.4309x over previous
import functools

import jax
import jax.numpy as jnp
from jax import lax
from jax.experimental import pallas as pl
from jax.experimental.pallas import tpu as pltpu
from jax.experimental.pallas import tpu_sc as plsc

N = 10000
E = 160000
L = 128
ITERS = 2

NC = 2
NS = 16
CH = 128
NCH = 79
EPS = CH * NCH
E_PAD = EPS * NS
NROWS = 10112
ZPS = NROWS // NS

_f32 = jnp.float32


def _sc_phi(tA, tB, cc, gidx, ia, ib, isc, W):
    mesh = plsc.VectorSubcoreMesh(core_axis_name="c", subcore_axis_name="s",
                                  num_cores=NC, num_subcores=NS)
    nv = W // 16

    @functools.partial(
        pl.kernel,
        out_type=jax.ShapeDtypeStruct((2, NROWS, W), _f32),
        mesh=mesh,
        compiler_params=pltpu.CompilerParams(use_tc_tiling_on_sc=False),
        scratch_types=[
            pltpu.VMEM((CH, W), _f32),
            pltpu.VMEM((CH, W), _f32),
            pltpu.VMEM((CH, W), _f32),
            pltpu.VMEM((CH,), jnp.int32),
            pltpu.VMEM((CH,), jnp.int32),
            pltpu.VMEM((CH,), jnp.int32),
            pltpu.VMEM_SHARED((NROWS, W), _f32),
            pltpu.SemaphoreType.DMA,
        ],
    )
    def body(tAr, tBr, ccr, gidxr, outr,
             bufA, bufB, bufC, idxA, idxB, sidx, acc, sem):
        cid = lax.axis_index("c")
        sid = lax.axis_index("s")

        @pl.loop(0, CH)
        def _(i):
            for j in range(nv):
                bufA[i, pl.ds(j * 16, 16)] = jnp.zeros((16,), _f32)

        zbase = sid * ZPS
        for k in range(4):
            pltpu.sync_copy(bufA.at[pl.ds(0, CH)], acc.at[pl.ds(zbase + k * CH, CH)])
        pltpu.sync_copy(bufA.at[pl.ds(0, ZPS - 4 * CH)],
                        acc.at[pl.ds(zbase + 4 * CH, ZPS - 4 * CH)])
        plsc.subcore_barrier()

        @pl.loop(0, NCH)
        def _(g):
            base = sid * EPS + g * CH
            pltpu.sync_copy(gidxr.at[ia].at[pl.ds(base, CH)], idxA)
            pltpu.sync_copy(gidxr.at[ib].at[pl.ds(base, CH)], idxB)
            pltpu.sync_copy(gidxr.at[isc].at[pl.ds(base, CH)], sidx)

            def gathers(c):
                cps = [
                    pltpu.async_copy(tAr.at[c].at[idxA], bufA, sem),
                    pltpu.async_copy(tBr.at[c].at[idxB], bufB, sem),
                    pltpu.async_copy(ccr.at[c].at[pl.ds(base, CH)], bufC, sem),
                ]
                for d in cps:
                    d.wait()

            @pl.when(cid == 0)
            def _():
                gathers(0)

            @pl.when(cid == 1)
            def _():
                gathers(1)

            @plsc.parallel_loop(0, CH, unroll=2)
            def _(i):
                for j in range(nv):
                    sl = pl.ds(j * 16, 16)
                    v = bufA[i, sl] + bufB[i, sl] + bufC[i, sl]
                    bufA[i, sl] = jnp.maximum(v, 0.0)

            pltpu.sync_copy(bufA, acc.at[sidx], add=True)

        plsc.subcore_barrier()

        def copy_out(c):
            for k in range(5):
                nrow = CH if k < 4 else ZPS - 4 * CH
                r = zbase + k * CH
                pltpu.sync_copy(acc.at[pl.ds(r, nrow)],
                                outr.at[c].at[pl.ds(r, nrow)])

        @pl.when(cid == 0)
        def _():
            copy_out(0)

        @pl.when(cid == 1)
        def _():
            copy_out(1)

    return body(tA, tB, cc, gidx)


def _sc_degrees(gidx):
    mesh = plsc.VectorSubcoreMesh(core_axis_name="c", subcore_axis_name="s",
                                  num_cores=NC, num_subcores=NS)

    @functools.partial(
        pl.kernel,
        out_type=jax.ShapeDtypeStruct((2, NROWS, 16), _f32),
        mesh=mesh,
        compiler_params=pltpu.CompilerParams(use_tc_tiling_on_sc=False),
        scratch_types=[
            pltpu.VMEM((CH, 16), _f32),
            pltpu.VMEM((CH, 16), _f32),
            pltpu.VMEM((CH,), jnp.int32),
            pltpu.VMEM_SHARED((NROWS, 16), _f32),
        ],
    )
    def body(gidxr, degr, ones, zer, idx, acc):
        cid = lax.axis_index("c")
        sid = lax.axis_index("s")

        @pl.loop(0, CH)
        def _(i):
            ones[i, pl.ds(0, 16)] = jnp.ones((16,), _f32)
            zer[i, pl.ds(0, 16)] = jnp.zeros((16,), _f32)

        zbase = sid * ZPS
        for k in range(4):
            pltpu.sync_copy(zer.at[pl.ds(0, CH)], acc.at[pl.ds(zbase + k * CH, CH)])
        pltpu.sync_copy(zer.at[pl.ds(0, ZPS - 4 * CH)],
                        acc.at[pl.ds(zbase + 4 * CH, ZPS - 4 * CH)])
        plsc.subcore_barrier()

        @pl.loop(0, NCH)
        def _(g):
            base = sid * EPS + g * CH

            @pl.when(cid == 0)
            def _():
                pltpu.sync_copy(gidxr.at[2].at[pl.ds(base, CH)], idx)

            @pl.when(cid == 1)
            def _():
                pltpu.sync_copy(gidxr.at[3].at[pl.ds(base, CH)], idx)

            pltpu.sync_copy(ones, acc.at[idx], add=True)

        plsc.subcore_barrier()

        def copy_out(c):
            for k in range(5):
                nrow = CH if k < 4 else ZPS - 4 * CH
                r = zbase + k * CH
                pltpu.sync_copy(acc.at[pl.ds(r, nrow)], degr.at[c].at[pl.ds(r, nrow)])

        @pl.when(cid == 0)
        def _():
            copy_out(0)

        @pl.when(cid == 1)
        def _():
            copy_out(1)

    return body(gidx)


_NB = 2000
_EB = 1024


def _full(shape):
    return pl.BlockSpec(shape, lambda i: tuple(0 for _ in shape))


def _tc_edge_consts(ea8, wto, wfr, wnm):
    def body(ea_r, wto_r, wfr_r, wnm_r, c64_r, c128_r):
        ea = ea_r[...]
        cto = jnp.dot(ea, wto_r[...], preferred_element_type=_f32)
        cfr = jnp.dot(ea, wfr_r[...], preferred_element_type=_f32)
        cnm = jnp.dot(ea, wnm_r[...], preferred_element_type=_f32)
        c64_r[0, :, :] = cto[:, :64]
        c64_r[1, :, :] = cto[:, 64:]
        c128_r[0, :, :] = jnp.concatenate([cfr[:, :64], cnm[:, :64]], axis=1)
        c128_r[1, :, :] = jnp.concatenate([cfr[:, 64:], cnm[:, 64:]], axis=1)

    return pl.pallas_call(
        body,
        grid=(E_PAD // _EB,),
        in_specs=[pl.BlockSpec((_EB, 8), lambda i: (i, 0)),
                  _full((8, 128)), _full((8, 128)), _full((8, 128))],
        out_specs=[pl.BlockSpec((2, _EB, 64), lambda i: (0, i, 0)),
                   pl.BlockSpec((2, _EB, 128), lambda i: (0, i, 0))],
        out_shape=[jax.ShapeDtypeStruct((2, E_PAD, 64), _f32),
                   jax.ShapeDtypeStruct((2, E_PAD, 128), _f32)],
    )(ea8, wto, wfr, wnm)


def _proj_out_specs():
    return [pl.BlockSpec((2, _NB, 64), lambda i: (0, i, 0)),
            pl.BlockSpec((2, _NB, 64), lambda i: (0, i, 0)),
            pl.BlockSpec((2, _NB, 128), lambda i: (0, i, 0)),
            pl.BlockSpec((2, _NB, 128), lambda i: (0, i, 0))]


def _proj_out_shapes():
    return [jax.ShapeDtypeStruct((2, N, 64), _f32),
            jax.ShapeDtypeStruct((2, N, 64), _f32),
            jax.ShapeDtypeStruct((2, N, 128), _f32),
            jax.ShapeDtypeStruct((2, N, 128), _f32)]


def _write_proj(h, wproj_r, toA_r, toB_r, fnA_r, fnB_r):
    proj = jnp.dot(h, wproj_r[...], preferred_element_type=_f32)
    toA_r[0, :, :] = proj[:, 0:64]
    toA_r[1, :, :] = proj[:, 64:128]
    toB_r[0, :, :] = proj[:, 128:192]
    toB_r[1, :, :] = proj[:, 192:256]
    fnA_r[0, :, :] = proj[:, 256:384]
    fnA_r[1, :, :] = proj[:, 384:512]
    fnB_r[0, :, :] = proj[:, 512:640]
    fnB_r[1, :, :] = proj[:, 640:768]


def _tc_encoder(xp, encv, encW2, wproj):
    def body(xp_r, encv_r, encW2_r, wproj_r, h_r, toA_r, toB_r, fnA_r, fnB_r):
        x0 = xp_r[:, 0:1]
        h1 = jax.nn.relu(x0 * encv_r[0:1, :] + encv_r[1:2, :])
        h = jnp.dot(h1, encW2_r[...], preferred_element_type=_f32) + encv_r[2:3, :]
        h_r[...] = h
        _write_proj(h, wproj_r, toA_r, toB_r, fnA_r, fnB_r)

    return pl.pallas_call(
        body,
        grid=(N // _NB,),
        in_specs=[pl.BlockSpec((_NB, 128), lambda i: (i, 0)),
                  _full((8, 128)), _full((128, 128)), _full((128, 768))],
        out_specs=[pl.BlockSpec((_NB, 128), lambda i: (i, 0))] + _proj_out_specs(),
        out_shape=[jax.ShapeDtypeStruct((N, 128), _f32)] + _proj_out_shapes(),
    )(xp, encv, encW2, wproj)


def _tc_update(h, h_init, s_to, s_fn, extras, W):
    def body(h_r, hi_r, sto_r, sfn_r, ex_r,
             toW2_r, frW2_r, nmW2_r, upW1h_r, upW1t_r, upW1f_r, upx_r, upW2_r,
             unW1h_r, unW1n_r, unx_r, unW2_r, alpack_r, vecs_r, wproj_r,
             hn_r, toA_r, toB_r, fnA_r, fnB_r):
        hb = h_r[...]
        ex = ex_r[...]
        dot = lambda a, b: jnp.dot(a, b, preferred_element_type=_f32)
        mp_to = (dot(sto_r[0, :, :], toW2_r[0:64, :]) +
                 dot(sto_r[1, :, :], toW2_r[64:128, :]) + ex[:, 8:9] * vecs_r[0:1, :])
        mp_fr = (dot(sfn_r[0, :, 0:64], frW2_r[0:64, :]) +
                 dot(sfn_r[1, :, 0:64], frW2_r[64:128, :]) + ex[:, 9:10] * vecs_r[1:2, :])
        mp_nm = (dot(sfn_r[0, :, 64:128], nmW2_r[0:64, :]) +
                 dot(sfn_r[1, :, 64:128], nmW2_r[64:128, :]) + ex[:, 9:10] * vecs_r[2:3, :])

        logit = jnp.sum(hb * alpack_r[0:1, :] + mp_to * alpack_r[1:2, :] +
                        mp_fr * alpack_r[2:3, :] + ex * alpack_r[3:4, :],
                        axis=1, keepdims=True)
        alpha = jax.nn.sigmoid(logit)
        u1 = jax.nn.relu(dot(hb, upW1h_r[...]) + dot(mp_to, upW1t_r[...]) +
                         dot(mp_fr, upW1f_r[...]) + dot(ex, upx_r[...]))
        upd_int = alpha * (dot(u1, upW2_r[...]) + vecs_r[3:4, :])
        n1 = jax.nn.relu(dot(hb, unW1h_r[...]) + dot(mp_nm, unW1n_r[...]) +
                         dot(ex, unx_r[...]))
        upd_neu = dot(n1, unW2_r[...]) + vecs_r[4:5, :]

        hn = hb + upd_int
        hn = jnp.where(ex[:, 7:8] > 0.5, upd_neu, hn)
        mu = jnp.mean(hn, axis=1, keepdims=True)
        var = jnp.mean((hn - mu) * (hn - mu), axis=1, keepdims=True)
        hn = (hn - mu) * lax.rsqrt(var + 1e-5) * vecs_r[5:6, :] + vecs_r[6:7, :]
        hn = jnp.where(ex[:, 6:7] > 0.5, hi_r[...], hn)
        hn_r[...] = hn
        _write_proj(hn, wproj_r, toA_r, toB_r, fnA_r, fnB_r)

    nb = pl.BlockSpec((_NB, 128), lambda i: (i, 0))
    return pl.pallas_call(
        body,
        grid=(N // _NB,),
        in_specs=[nb, nb,
                  pl.BlockSpec((2, _NB, 64), lambda i: (0, i, 0)),
                  pl.BlockSpec((2, _NB, 128), lambda i: (0, i, 0)),
                  nb,
                  _full((128, 128)), _full((128, 128)), _full((128, 128)),
                  _full((128, 128)), _full((128, 128)), _full((128, 128)),
                  _full((128, 128)), _full((128, 128)), _full((128, 128)),
                  _full((128, 128)), _full((128, 128)), _full((128, 128)),
                  _full((8, 128)), _full((8, 128)), _full((128, 768))],
        out_specs=[nb] + _proj_out_specs(),
        out_shape=[jax.ShapeDtypeStruct((N, 128), _f32)] + _proj_out_shapes(),
    )(h, h_init, s_to, s_fn, extras, *W)


def _tc_decoder(h, decW1, decv):
    def body(h_r, decW1_r, decv_r, out_r):
        u1 = jax.nn.relu(jnp.dot(h_r[...], decW1_r[...],
                                 preferred_element_type=_f32) + decv_r[0:1, :])
        val = jnp.sum(u1 * decv_r[1:2, :], axis=1, keepdims=True) + decv_r[2:3, 0:1]
        out_r[...] = jnp.broadcast_to(val, (_NB, 128))

    return pl.pallas_call(
        body,
        grid=(N // _NB,),
        in_specs=[pl.BlockSpec((_NB, 128), lambda i: (i, 0)),
                  _full((128, 128)), _full((8, 128))],
        out_specs=pl.BlockSpec((_NB, 128), lambda i: (i, 0)),
        out_shape=jax.ShapeDtypeStruct((N, 128), _f32),
    )(h, decW1, decv)


def kernel(x, edge_index, edge_attr, prb_data, unit_normal_vector, tags, params):
    p = params
    src = edge_index[0]
    dst = edge_index[1]
    keep = src != dst
    dst_m = jnp.where(keep, dst, N)
    src_m = jnp.where(keep, src, N)

    pad = E_PAD - E
    gidx = jnp.stack([
        jnp.pad(dst, (0, pad)),
        jnp.pad(src, (0, pad)),
        jnp.pad(dst_m, (0, pad), constant_values=N),
        jnp.pad(src_m, (0, pad), constant_values=N),
    ]).astype(jnp.int32)

    ea8 = jnp.zeros((E_PAD, 8), _f32)
    ea8 = ea8.at[:E, 0:3].set(edge_attr)
    ea8 = ea8.at[:E, 3].set(1.0)

    def wpack(W1, b1):
        w = jnp.zeros((8, 128), _f32)
        return w.at[0:3, :].set(W1[256:259]).at[3, :].set(b1)

    wto = wpack(p['to_W1'], p['to_b1'])
    wfr = wpack(p['fr_W1'], p['fr_b1'])
    wnm = wpack(p['nm_W1'], p['nm_b1'])

    toA = p['to_W1'][:128]
    toB = p['to_W1'][128:256]
    frA, frB = p['fr_W1'][:128], p['fr_W1'][128:256]
    nmA, nmB = p['nm_W1'][:128], p['nm_W1'][128:256]
    wproj = jnp.concatenate([
        toA[:, :64], toA[:, 64:], toB[:, :64], toB[:, 64:],
        frA[:, :64], nmA[:, :64], frA[:, 64:], nmA[:, 64:],
        frB[:, :64], nmB[:, :64], frB[:, 64:], nmB[:, 64:],
    ], axis=1)

    encv = jnp.zeros((8, 128), _f32)
    encv = encv.at[0].set(p['enc_W1'][0]).at[1].set(p['enc_b1']).at[2].set(p['enc_b2'])
    decv = jnp.zeros((8, 128), _f32)
    decv = decv.at[0].set(p['dec_b1']).at[1].set(p['dec_W2'][:, 0]).at[2, 0].set(p['dec_b2'][0])

    alpack = jnp.zeros((8, 128), _f32)
    alpack = (alpack.at[0].set(p['al_W'][0:128, 0])
              .at[1].set(p['al_W'][128:256, 0])
              .at[2].set(p['al_W'][256:384, 0])
              .at[3, 0:3].set(p['al_W'][384:387, 0])
              .at[3, 5].set(p['al_b'][0]))
    upx = jnp.zeros((128, 128), _f32)
    upx = upx.at[0:3].set(p['up_W1'][384:387]).at[5].set(p['up_b1'])
    unx = jnp.zeros((128, 128), _f32)
    unx = (unx.at[0:3].set(p['un_W1'][256:259])
           .at[3:5].set(p['un_W1'][259:261]).at[5].set(p['un_b1']))
    vecs = jnp.stack([p['to_b2'], p['fr_b2'], p['nm_b2'], p['up_b2'],
                      p['un_b2'], p['ln_g'], p['ln_b'], jnp.zeros((128,), _f32)])

    W = (p['to_W2'], p['fr_W2'], p['nm_W2'],
         p['up_W1'][0:128], p['up_W1'][128:256], p['up_W1'][256:384], upx, p['up_W2'],
         p['un_W1'][0:128], p['un_W1'][128:256], unx, p['un_W2'],
         alpack, vecs, wproj)

    c64, c128 = _tc_edge_consts(ea8, wto, wfr, wnm)
    degs = _sc_degrees(gidx)

    extras = jnp.zeros((N, 128), _f32)
    extras = (extras.at[:, 0:3].set(prb_data)
              .at[:, 3:5].set(unit_normal_vector)
              .at[:, 5].set(1.0)
              .at[:, 6].set((tags[:, 1] == 1).astype(_f32))
              .at[:, 7].set((tags[:, 2] == 1).astype(_f32))
              .at[:, 8].set(degs[0, :N, 0])
              .at[:, 9].set(degs[1, :N, 0]))

    xp = jnp.zeros((N, 128), _f32).at[:, 0:1].set(x)
    h_init, toAt, toBt, fnAt, fnBt = _tc_encoder(xp, encv, p['enc_W2'], wproj)

    h = h_init
    for _ in range(ITERS):
        s_to = _sc_phi(toAt, toBt, c64, gidx, 0, 1, 2, 64)
        s_fn = _sc_phi(fnAt, fnBt, c128, gidx, 1, 0, 3, 128)
        h, toAt, toBt, fnAt, fnBt = _tc_update(h, h_init, s_to, s_fn, extras, W)

    out = _tc_decoder(h, p['dec_W1'], decv)
    return out[:, 0:1]


# speedup vs baseline: 1.4494x; 1.0129x over previous
import functools

import jax
import jax.numpy as jnp
from jax import lax
from jax.experimental import pallas as pl
from jax.experimental.pallas import tpu as pltpu
from jax.experimental.pallas import tpu_sc as plsc

N = 10000
E = 160000
L = 128
ITERS = 2

NC = 2
NS = 16
CH = 128
NCH = 80
EPS = CH * NCH
E_PAD = EPS * NS
NT = E_PAD // CH
NROWS = 10112
ZPS = NROWS // NS

_f32 = jnp.float32


def _sc_phi(tA, tB, cc, g3, W, ch, nch):
    mesh = plsc.VectorSubcoreMesh(core_axis_name="c", subcore_axis_name="s",
                                  num_cores=NC, num_subcores=NS)
    nv = W // 16
    nz = ZPS // ch
    rz = ZPS - nz * ch

    @functools.partial(
        pl.kernel,
        out_type=jax.ShapeDtypeStruct((2, NROWS, W), _f32),
        mesh=mesh,
        compiler_params=pltpu.CompilerParams(use_tc_tiling_on_sc=False),
        scratch_types=[
            pltpu.VMEM((ch, W), _f32), pltpu.VMEM((ch, W), _f32),
            pltpu.VMEM((ch, W), _f32),
            pltpu.VMEM((ch, W), _f32), pltpu.VMEM((ch, W), _f32),
            pltpu.VMEM((ch, W), _f32),
            pltpu.VMEM((3, ch), jnp.int32),
            pltpu.VMEM((3, ch), jnp.int32),
            pltpu.VMEM_SHARED((NROWS, W), _f32),
            pltpu.SemaphoreType.DMA, pltpu.SemaphoreType.DMA,
            pltpu.SemaphoreType.DMA,
        ],
    )
    def body(tAr, tBr, ccr, g3r, outr,
             bufA0, bufB0, bufC0, bufA1, bufB1, bufC1, idx0, idx1,
             acc, sem0, sem1, semS):
        cid = lax.axis_index("c")
        sid = lax.axis_index("s")
        bufs = ((bufA0, bufB0, bufC0, idx0, sem0),
                (bufA1, bufB1, bufC1, idx1, sem1))

        @pl.loop(0, ch)
        def _(i):
            for j in range(nv):
                bufA0[i, pl.ds(j * 16, 16)] = jnp.zeros((16,), _f32)

        zbase = sid * ZPS
        for k in range(nz):
            pltpu.sync_copy(bufA0.at[pl.ds(0, ch)], acc.at[pl.ds(zbase + k * ch, ch)])
        if rz:
            pltpu.sync_copy(bufA0.at[pl.ds(0, rz)],
                            acc.at[pl.ds(zbase + nz * ch, rz)])
        plsc.subcore_barrier()

        def fire_idx(p, t):
            bA, bB, bC, idx, sem = bufs[p]
            return pltpu.async_copy(g3r.at[t], idx, sem)

        def fire_gathers(p, t):
            bA, bB, bC, idx, sem = bufs[p]
            return [pltpu.async_copy(tAr.at[cid].at[idx.at[0]], bA, sem),
                    pltpu.async_copy(tBr.at[cid].at[idx.at[1]], bB, sem),
                    pltpu.async_copy(ccr.at[cid].at[pl.ds(t * ch, ch)], bC, sem)]

        def relu_combine(p):
            bA, bB, bC, idx, sem = bufs[p]

            @plsc.parallel_loop(0, ch, unroll=2)
            def _(i):
                for j in range(nv):
                    sl = pl.ds(j * 16, 16)
                    bA[i, sl] = jnp.maximum(bA[i, sl] + bB[i, sl] + bC[i, sl], 0.0)

        def fire_scatter(p):
            bA, bB, bC, idx, sem = bufs[p]
            return pltpu.async_copy(bA, acc.at[idx.at[2]], semS, add=True)

        @pl.loop(0, nch // 2)
        def _(k):
            t0 = sid * nch + 2 * k
            t1 = t0 + 1
            di0 = fire_idx(0, t0)
            di1 = fire_idx(1, t1)
            di0.wait()
            g0 = fire_gathers(0, t0)
            di1.wait()
            g1 = fire_gathers(1, t1)
            for d in g0:
                d.wait()
            relu_combine(0)
            s0 = fire_scatter(0)
            for d in g1:
                d.wait()
            relu_combine(1)
            s1 = fire_scatter(1)
            s0.wait()
            s1.wait()

        plsc.subcore_barrier()

        for k in range(nz):
            r = zbase + k * ch
            pltpu.sync_copy(acc.at[pl.ds(r, ch)], outr.at[cid].at[pl.ds(r, ch)])
        if rz:
            r = zbase + nz * ch
            pltpu.sync_copy(acc.at[pl.ds(r, rz)], outr.at[cid].at[pl.ds(r, rz)])

    return body(tA, tB, cc, g3)


def _sc_degrees(gidx):
    mesh = plsc.VectorSubcoreMesh(core_axis_name="c", subcore_axis_name="s",
                                  num_cores=NC, num_subcores=NS)

    @functools.partial(
        pl.kernel,
        out_type=jax.ShapeDtypeStruct((2, NROWS, 16), _f32),
        mesh=mesh,
        compiler_params=pltpu.CompilerParams(use_tc_tiling_on_sc=False),
        scratch_types=[
            pltpu.VMEM((CH, 16), _f32),
            pltpu.VMEM((CH, 16), _f32),
            pltpu.VMEM((CH,), jnp.int32),
            pltpu.VMEM_SHARED((NROWS, 16), _f32),
        ],
    )
    def body(gidxr, degr, ones, zer, idx, acc):
        cid = lax.axis_index("c")
        sid = lax.axis_index("s")

        @pl.loop(0, CH)
        def _(i):
            ones[i, pl.ds(0, 16)] = jnp.ones((16,), _f32)
            zer[i, pl.ds(0, 16)] = jnp.zeros((16,), _f32)

        zbase = sid * ZPS
        for k in range(4):
            pltpu.sync_copy(zer.at[pl.ds(0, CH)], acc.at[pl.ds(zbase + k * CH, CH)])
        pltpu.sync_copy(zer.at[pl.ds(0, ZPS - 4 * CH)],
                        acc.at[pl.ds(zbase + 4 * CH, ZPS - 4 * CH)])
        plsc.subcore_barrier()

        @pl.loop(0, NCH)
        def _(g):
            base = sid * EPS + g * CH

            @pl.when(cid == 0)
            def _():
                pltpu.sync_copy(gidxr.at[2].at[pl.ds(base, CH)], idx)

            @pl.when(cid == 1)
            def _():
                pltpu.sync_copy(gidxr.at[3].at[pl.ds(base, CH)], idx)

            pltpu.sync_copy(ones, acc.at[idx], add=True)

        plsc.subcore_barrier()

        def copy_out(c):
            for k in range(5):
                nrow = CH if k < 4 else ZPS - 4 * CH
                r = zbase + k * CH
                pltpu.sync_copy(acc.at[pl.ds(r, nrow)], degr.at[c].at[pl.ds(r, nrow)])

        @pl.when(cid == 0)
        def _():
            copy_out(0)

        @pl.when(cid == 1)
        def _():
            copy_out(1)

    return body(gidx)


_NB = 2000
_EB = 1024


def _full(shape):
    return pl.BlockSpec(shape, lambda i: tuple(0 for _ in shape))


def _tc_edge_consts(ea8, wto, wfr, wnm):
    def body(ea_r, wto_r, wfr_r, wnm_r, c64_r, c128_r):
        ea = ea_r[...]
        cto = jnp.dot(ea, wto_r[...], preferred_element_type=_f32)
        cfr = jnp.dot(ea, wfr_r[...], preferred_element_type=_f32)
        cnm = jnp.dot(ea, wnm_r[...], preferred_element_type=_f32)
        c64_r[0, :, :] = cto[:, :64]
        c64_r[1, :, :] = cto[:, 64:]
        c128_r[0, :, :] = jnp.concatenate([cfr[:, :64], cnm[:, :64]], axis=1)
        c128_r[1, :, :] = jnp.concatenate([cfr[:, 64:], cnm[:, 64:]], axis=1)

    return pl.pallas_call(
        body,
        grid=(E_PAD // _EB,),
        in_specs=[pl.BlockSpec((_EB, 8), lambda i: (i, 0)),
                  _full((8, 128)), _full((8, 128)), _full((8, 128))],
        out_specs=[pl.BlockSpec((2, _EB, 64), lambda i: (0, i, 0)),
                   pl.BlockSpec((2, _EB, 128), lambda i: (0, i, 0))],
        out_shape=[jax.ShapeDtypeStruct((2, E_PAD, 64), _f32),
                   jax.ShapeDtypeStruct((2, E_PAD, 128), _f32)],
    )(ea8, wto, wfr, wnm)


def _proj_out_specs():
    return [pl.BlockSpec((2, _NB, 64), lambda i: (0, i, 0)),
            pl.BlockSpec((2, _NB, 64), lambda i: (0, i, 0)),
            pl.BlockSpec((2, _NB, 128), lambda i: (0, i, 0)),
            pl.BlockSpec((2, _NB, 128), lambda i: (0, i, 0))]


def _proj_out_shapes():
    return [jax.ShapeDtypeStruct((2, N, 64), _f32),
            jax.ShapeDtypeStruct((2, N, 64), _f32),
            jax.ShapeDtypeStruct((2, N, 128), _f32),
            jax.ShapeDtypeStruct((2, N, 128), _f32)]


def _write_proj(h, wproj_r, toA_r, toB_r, fnA_r, fnB_r):
    proj = jnp.dot(h, wproj_r[...], preferred_element_type=_f32)
    toA_r[0, :, :] = proj[:, 0:64]
    toA_r[1, :, :] = proj[:, 64:128]
    toB_r[0, :, :] = proj[:, 128:192]
    toB_r[1, :, :] = proj[:, 192:256]
    fnA_r[0, :, :] = proj[:, 256:384]
    fnA_r[1, :, :] = proj[:, 384:512]
    fnB_r[0, :, :] = proj[:, 512:640]
    fnB_r[1, :, :] = proj[:, 640:768]


def _tc_encoder(xp, encv, encW2, wproj):
    def body(xp_r, encv_r, encW2_r, wproj_r, h_r, toA_r, toB_r, fnA_r, fnB_r):
        x0 = xp_r[:, 0:1]
        h1 = jax.nn.relu(x0 * encv_r[0:1, :] + encv_r[1:2, :])
        h = jnp.dot(h1, encW2_r[...], preferred_element_type=_f32) + encv_r[2:3, :]
        h_r[...] = h
        _write_proj(h, wproj_r, toA_r, toB_r, fnA_r, fnB_r)

    return pl.pallas_call(
        body,
        grid=(N // _NB,),
        in_specs=[pl.BlockSpec((_NB, 128), lambda i: (i, 0)),
                  _full((8, 128)), _full((128, 128)), _full((128, 768))],
        out_specs=[pl.BlockSpec((_NB, 128), lambda i: (i, 0))] + _proj_out_specs(),
        out_shape=[jax.ShapeDtypeStruct((N, 128), _f32)] + _proj_out_shapes(),
    )(xp, encv, encW2, wproj)


def _tc_update(h, h_init, s_to, s_fn, extras, W):
    def body(h_r, hi_r, sto_r, sfn_r, ex_r,
             toW2_r, frW2_r, nmW2_r, upW1h_r, upW1t_r, upW1f_r, upx_r, upW2_r,
             unW1h_r, unW1n_r, unx_r, unW2_r, alpack_r, vecs_r, wproj_r,
             hn_r, toA_r, toB_r, fnA_r, fnB_r):
        hb = h_r[...]
        ex = ex_r[...]
        dot = lambda a, b: jnp.dot(a, b, preferred_element_type=_f32)
        mp_to = (dot(sto_r[0, :, :], toW2_r[0:64, :]) +
                 dot(sto_r[1, :, :], toW2_r[64:128, :]) + ex[:, 8:9] * vecs_r[0:1, :])
        mp_fr = (dot(sfn_r[0, :, 0:64], frW2_r[0:64, :]) +
                 dot(sfn_r[1, :, 0:64], frW2_r[64:128, :]) + ex[:, 9:10] * vecs_r[1:2, :])
        mp_nm = (dot(sfn_r[0, :, 64:128], nmW2_r[0:64, :]) +
                 dot(sfn_r[1, :, 64:128], nmW2_r[64:128, :]) + ex[:, 9:10] * vecs_r[2:3, :])

        logit = jnp.sum(hb * alpack_r[0:1, :] + mp_to * alpack_r[1:2, :] +
                        mp_fr * alpack_r[2:3, :] + ex * alpack_r[3:4, :],
                        axis=1, keepdims=True)
        alpha = jax.nn.sigmoid(logit)
        u1 = jax.nn.relu(dot(hb, upW1h_r[...]) + dot(mp_to, upW1t_r[...]) +
                         dot(mp_fr, upW1f_r[...]) + dot(ex, upx_r[...]))
        upd_int = alpha * (dot(u1, upW2_r[...]) + vecs_r[3:4, :])
        n1 = jax.nn.relu(dot(hb, unW1h_r[...]) + dot(mp_nm, unW1n_r[...]) +
                         dot(ex, unx_r[...]))
        upd_neu = dot(n1, unW2_r[...]) + vecs_r[4:5, :]

        hn = hb + upd_int
        hn = jnp.where(ex[:, 7:8] > 0.5, upd_neu, hn)
        mu = jnp.mean(hn, axis=1, keepdims=True)
        var = jnp.mean((hn - mu) * (hn - mu), axis=1, keepdims=True)
        hn = (hn - mu) * lax.rsqrt(var + 1e-5) * vecs_r[5:6, :] + vecs_r[6:7, :]
        hn = jnp.where(ex[:, 6:7] > 0.5, hi_r[...], hn)
        hn_r[...] = hn
        _write_proj(hn, wproj_r, toA_r, toB_r, fnA_r, fnB_r)

    nb = pl.BlockSpec((_NB, 128), lambda i: (i, 0))
    return pl.pallas_call(
        body,
        grid=(N // _NB,),
        in_specs=[nb, nb,
                  pl.BlockSpec((2, _NB, 64), lambda i: (0, i, 0)),
                  pl.BlockSpec((2, _NB, 128), lambda i: (0, i, 0)),
                  nb,
                  _full((128, 128)), _full((128, 128)), _full((128, 128)),
                  _full((128, 128)), _full((128, 128)), _full((128, 128)),
                  _full((128, 128)), _full((128, 128)), _full((128, 128)),
                  _full((128, 128)), _full((128, 128)), _full((128, 128)),
                  _full((8, 128)), _full((8, 128)), _full((128, 768))],
        out_specs=[nb] + _proj_out_specs(),
        out_shape=[jax.ShapeDtypeStruct((N, 128), _f32)] + _proj_out_shapes(),
    )(h, h_init, s_to, s_fn, extras, *W)


def _tc_decoder(h, decW1, decv):
    def body(h_r, decW1_r, decv_r, out_r):
        u1 = jax.nn.relu(jnp.dot(h_r[...], decW1_r[...],
                                 preferred_element_type=_f32) + decv_r[0:1, :])
        val = jnp.sum(u1 * decv_r[1:2, :], axis=1, keepdims=True) + decv_r[2:3, 0:1]
        out_r[...] = jnp.broadcast_to(val, (_NB, 128))

    return pl.pallas_call(
        body,
        grid=(N // _NB,),
        in_specs=[pl.BlockSpec((_NB, 128), lambda i: (i, 0)),
                  _full((128, 128)), _full((8, 128))],
        out_specs=pl.BlockSpec((_NB, 128), lambda i: (i, 0)),
        out_shape=jax.ShapeDtypeStruct((N, 128), _f32),
    )(h, decW1, decv)


def kernel(x, edge_index, edge_attr, prb_data, unit_normal_vector, tags, params):
    p = params
    src = edge_index[0]
    dst = edge_index[1]
    keep = src != dst
    dst_m = jnp.where(keep, dst, N)
    src_m = jnp.where(keep, src, N)

    pad = E_PAD - E
    dst_p = jnp.pad(dst, (0, pad)).astype(jnp.int32)
    src_p = jnp.pad(src, (0, pad)).astype(jnp.int32)
    dstm_p = jnp.pad(dst_m, (0, pad), constant_values=N).astype(jnp.int32)
    srcm_p = jnp.pad(src_m, (0, pad), constant_values=N).astype(jnp.int32)
    gidx = jnp.stack([dst_p, src_p, dstm_p, srcm_p])
    g3to = jnp.stack([dst_p, src_p, dstm_p]).reshape(3, E_PAD // 128, 128).transpose(1, 0, 2)
    g3fn = jnp.stack([src_p, dst_p, srcm_p]).reshape(3, E_PAD // 64, 64).transpose(1, 0, 2)

    ea8 = jnp.zeros((E_PAD, 8), _f32)
    ea8 = ea8.at[:E, 0:3].set(edge_attr)
    ea8 = ea8.at[:E, 3].set(1.0)

    def wpack(W1, b1):
        w = jnp.zeros((8, 128), _f32)
        return w.at[0:3, :].set(W1[256:259]).at[3, :].set(b1)

    wto = wpack(p['to_W1'], p['to_b1'])
    wfr = wpack(p['fr_W1'], p['fr_b1'])
    wnm = wpack(p['nm_W1'], p['nm_b1'])

    toA = p['to_W1'][:128]
    toB = p['to_W1'][128:256]
    frA, frB = p['fr_W1'][:128], p['fr_W1'][128:256]
    nmA, nmB = p['nm_W1'][:128], p['nm_W1'][128:256]
    wproj = jnp.concatenate([
        toA[:, :64], toA[:, 64:], toB[:, :64], toB[:, 64:],
        frA[:, :64], nmA[:, :64], frA[:, 64:], nmA[:, 64:],
        frB[:, :64], nmB[:, :64], frB[:, 64:], nmB[:, 64:],
    ], axis=1)

    encv = jnp.zeros((8, 128), _f32)
    encv = encv.at[0].set(p['enc_W1'][0]).at[1].set(p['enc_b1']).at[2].set(p['enc_b2'])
    decv = jnp.zeros((8, 128), _f32)
    decv = decv.at[0].set(p['dec_b1']).at[1].set(p['dec_W2'][:, 0]).at[2, 0].set(p['dec_b2'][0])

    alpack = jnp.zeros((8, 128), _f32)
    alpack = (alpack.at[0].set(p['al_W'][0:128, 0])
              .at[1].set(p['al_W'][128:256, 0])
              .at[2].set(p['al_W'][256:384, 0])
              .at[3, 0:3].set(p['al_W'][384:387, 0])
              .at[3, 5].set(p['al_b'][0]))
    upx = jnp.zeros((128, 128), _f32)
    upx = upx.at[0:3].set(p['up_W1'][384:387]).at[5].set(p['up_b1'])
    unx = jnp.zeros((128, 128), _f32)
    unx = (unx.at[0:3].set(p['un_W1'][256:259])
           .at[3:5].set(p['un_W1'][259:261]).at[5].set(p['un_b1']))
    vecs = jnp.stack([p['to_b2'], p['fr_b2'], p['nm_b2'], p['up_b2'],
                      p['un_b2'], p['ln_g'], p['ln_b'], jnp.zeros((128,), _f32)])

    W = (p['to_W2'], p['fr_W2'], p['nm_W2'],
         p['up_W1'][0:128], p['up_W1'][128:256], p['up_W1'][256:384], upx, p['up_W2'],
         p['un_W1'][0:128], p['un_W1'][128:256], unx, p['un_W2'],
         alpack, vecs, wproj)

    c64, c128 = _tc_edge_consts(ea8, wto, wfr, wnm)
    degs = _sc_degrees(gidx)

    extras = jnp.zeros((N, 128), _f32)
    extras = (extras.at[:, 0:3].set(prb_data)
              .at[:, 3:5].set(unit_normal_vector)
              .at[:, 5].set(1.0)
              .at[:, 6].set((tags[:, 1] == 1).astype(_f32))
              .at[:, 7].set((tags[:, 2] == 1).astype(_f32))
              .at[:, 8].set(degs[0, :N, 0])
              .at[:, 9].set(degs[1, :N, 0]))

    xp = jnp.zeros((N, 128), _f32).at[:, 0:1].set(x)
    h_init, toAt, toBt, fnAt, fnBt = _tc_encoder(xp, encv, p['enc_W2'], wproj)

    h = h_init
    for _ in range(ITERS):
        s_to = _sc_phi(toAt, toBt, c64, g3to, 64, 128, 80)
        s_fn = _sc_phi(fnAt, fnBt, c128, g3fn, 128, 64, 160)
        h, toAt, toBt, fnAt, fnBt = _tc_update(h, h_init, s_to, s_fn, extras, W)

    out = _tc_decoder(h, p['dec_W1'], decv)
    return out[:, 0:1]


# speedup vs baseline: 1.7520x; 1.2088x over previous
import functools

import jax
import jax.numpy as jnp
from jax import lax
from jax.experimental import pallas as pl
from jax.experimental.pallas import tpu as pltpu
from jax.experimental.pallas import tpu_sc as plsc

N = 10000
E = 160000
L = 128
ITERS = 2

NC = 2
NS = 16
CH = 128
NCH = 80
EPS = CH * NCH
E_PAD = EPS * NS
NT = E_PAD // CH
NROWS = 10112
ZPS = NROWS // NS
NRW = 10008
OB = 624
OW = 640

_f32 = jnp.float32


def _sc_phi(tA, tB, cc, g3, W, ch, nch):
    mesh = plsc.VectorSubcoreMesh(core_axis_name="c", subcore_axis_name="s",
                                  num_cores=NC, num_subcores=NS)
    nv = W // 16
    assert (nch - 4) % 4 == 0 and ch * nch == EPS

    @functools.partial(
        pl.kernel,
        out_type=jax.ShapeDtypeStruct((2, NRW, W), _f32),
        mesh=mesh,
        compiler_params=pltpu.CompilerParams(use_tc_tiling_on_sc=False),
        scratch_types=[
            pltpu.VMEM((ch, W), _f32), pltpu.VMEM((ch, W), _f32),
            pltpu.VMEM((ch, W), _f32),
            pltpu.VMEM((ch, W), _f32), pltpu.VMEM((ch, W), _f32),
            pltpu.VMEM((ch, W), _f32),
            pltpu.VMEM((3, ch), jnp.int32), pltpu.VMEM((3, ch), jnp.int32),
            pltpu.VMEM((3, ch), jnp.int32), pltpu.VMEM((3, ch), jnp.int32),
            pltpu.VMEM_SHARED((NRW, W), _f32),
            pltpu.SemaphoreType.DMA, pltpu.SemaphoreType.DMA,
            pltpu.SemaphoreType.DMA, pltpu.SemaphoreType.DMA,
            pltpu.SemaphoreType.DMA, pltpu.SemaphoreType.DMA,
            pltpu.SemaphoreType.DMA, pltpu.SemaphoreType.DMA,
        ],
    )
    def body(tAr, tBr, ccr, g3r, outr,
             bufA0, bufB0, bufC0, bufA1, bufB1, bufC1,
             ix0, ix1, ix2, ix3, acc,
             semG0, semG1, semS0, semS1, semI0, semI1, semI2, semI3):
        cid = lax.axis_index("c")
        sid = lax.axis_index("s")
        data = ((bufA0, bufB0, bufC0, semG0, semS0),
                (bufA1, bufB1, bufC1, semG1, semS1))
        ixs = ((ix0, semI0), (ix1, semI1), (ix2, semI2), (ix3, semI3))

        @pl.loop(0, ch)
        def _(i):
            for j in range(nv):
                bufA0[i, pl.ds(j * 16, 16)] = jnp.zeros((16,), _f32)

        obase = sid * OB
        for k in range(OW // ch):
            pltpu.sync_copy(bufA0.at[pl.ds(0, ch)],
                            acc.at[pl.ds(obase + k * ch, ch)])
        plsc.subcore_barrier()

        def fire_idx(s, t):
            ix, semI = ixs[s]
            pltpu.async_copy(g3r.at[t], ix, semI)

        def wI(s):
            ix, semI = ixs[s]
            pltpu.make_async_copy(g3r.at[0], ix, semI).wait()

        def fire_gathers(p, s, t):
            bA, bB, bC, semG, semS = data[p]
            ix, _ = ixs[s]
            pltpu.async_copy(tAr.at[cid].at[ix.at[0]], bA, semG)
            pltpu.async_copy(tBr.at[cid].at[ix.at[1]], bB, semG)
            pltpu.async_copy(ccr.at[cid].at[pl.ds(t * ch, ch)], bC, semG)

        def wG(p):
            bA, bB, bC, semG, semS = data[p]
            pltpu.make_async_copy(tAr.at[cid].at[ixs[0][0].at[0]], bA, semG).wait()
            pltpu.make_async_copy(tAr.at[cid].at[ixs[0][0].at[0]], bB, semG).wait()
            pltpu.make_async_copy(ccr.at[cid].at[pl.ds(0, ch)], bC, semG).wait()

        def relu_combine(p):
            bA, bB, bC, semG, semS = data[p]

            @plsc.parallel_loop(0, ch, unroll=2)
            def _(i):
                for j in range(nv):
                    sl = pl.ds(j * 16, 16)
                    bA[i, sl] = jnp.maximum(bA[i, sl] + bB[i, sl] + bC[i, sl], 0.0)

        def fire_scatter(p, s):
            bA, bB, bC, semG, semS = data[p]
            pltpu.async_copy(bA, acc.at[ixs[s][0].at[2]], semS, add=True)

        def wS(p):
            bA, bB, bC, semG, semS = data[p]
            pltpu.make_async_copy(bA, acc.at[ixs[0][0].at[2]], semS).wait()

        def step(p, s, t, *, do_ws=True, nxt=None, last=False):
            wG(p)
            if do_ws:
                wS(1 - p)
            if nxt is not None:
                fire_idx(nxt[0], nxt[1])
            if not last:
                wI((s + 1) % 4)
                fire_gathers(1 - p, (s + 1) % 4, t + 1)
            relu_combine(p)
            fire_scatter(p, s)

        T = sid * nch
        fire_idx(0, T)
        fire_idx(1, T + 1)
        wI(0)
        fire_gathers(0, 0, T)
        step(0, 0, T, do_ws=False, nxt=(2, T + 2))
        step(1, 1, T + 1, nxt=(3, T + 3))

        @pl.loop(0, (nch - 4) // 4)
        def _(j):
            c = T + 2 + 4 * j
            step(0, 2, c, nxt=(0, c + 2))
            step(1, 3, c + 1, nxt=(1, c + 3))
            step(0, 0, c + 2, nxt=(2, c + 4))
            step(1, 1, c + 3, nxt=(3, c + 5))

        step(0, 2, T + nch - 2)
        step(1, 3, T + nch - 1, last=True)
        wS(1)

        plsc.subcore_barrier()

        for k in range(OW // ch):
            r = obase + k * ch
            pltpu.sync_copy(acc.at[pl.ds(r, ch)], outr.at[cid].at[pl.ds(r, ch)])

    return body(tA, tB, cc, g3)


def _sc_degrees(gidx):
    mesh = plsc.VectorSubcoreMesh(core_axis_name="c", subcore_axis_name="s",
                                  num_cores=NC, num_subcores=NS)

    @functools.partial(
        pl.kernel,
        out_type=jax.ShapeDtypeStruct((2, NROWS, 16), _f32),
        mesh=mesh,
        compiler_params=pltpu.CompilerParams(use_tc_tiling_on_sc=False),
        scratch_types=[
            pltpu.VMEM((CH, 16), _f32),
            pltpu.VMEM((CH, 16), _f32),
            pltpu.VMEM((CH,), jnp.int32),
            pltpu.VMEM_SHARED((NROWS, 16), _f32),
        ],
    )
    def body(gidxr, degr, ones, zer, idx, acc):
        cid = lax.axis_index("c")
        sid = lax.axis_index("s")

        @pl.loop(0, CH)
        def _(i):
            ones[i, pl.ds(0, 16)] = jnp.ones((16,), _f32)
            zer[i, pl.ds(0, 16)] = jnp.zeros((16,), _f32)

        zbase = sid * ZPS
        for k in range(4):
            pltpu.sync_copy(zer.at[pl.ds(0, CH)], acc.at[pl.ds(zbase + k * CH, CH)])
        pltpu.sync_copy(zer.at[pl.ds(0, ZPS - 4 * CH)],
                        acc.at[pl.ds(zbase + 4 * CH, ZPS - 4 * CH)])
        plsc.subcore_barrier()

        @pl.loop(0, NCH)
        def _(g):
            base = sid * EPS + g * CH

            @pl.when(cid == 0)
            def _():
                pltpu.sync_copy(gidxr.at[2].at[pl.ds(base, CH)], idx)

            @pl.when(cid == 1)
            def _():
                pltpu.sync_copy(gidxr.at[3].at[pl.ds(base, CH)], idx)

            pltpu.sync_copy(ones, acc.at[idx], add=True)

        plsc.subcore_barrier()

        def copy_out(c):
            for k in range(5):
                nrow = CH if k < 4 else ZPS - 4 * CH
                r = zbase + k * CH
                pltpu.sync_copy(acc.at[pl.ds(r, nrow)], degr.at[c].at[pl.ds(r, nrow)])

        @pl.when(cid == 0)
        def _():
            copy_out(0)

        @pl.when(cid == 1)
        def _():
            copy_out(1)

    return body(gidx)


_NB = 2000
_EB = 1024


def _full(shape):
    return pl.BlockSpec(shape, lambda i: tuple(0 for _ in shape))


def _tc_edge_consts(ea8, wto, wfr, wnm):
    def body(ea_r, wto_r, wfr_r, wnm_r, c64_r, c128_r):
        ea = ea_r[...]
        cto = jnp.dot(ea, wto_r[...], preferred_element_type=_f32)
        cfr = jnp.dot(ea, wfr_r[...], preferred_element_type=_f32)
        cnm = jnp.dot(ea, wnm_r[...], preferred_element_type=_f32)
        c64_r[0, :, :] = cto[:, :64]
        c64_r[1, :, :] = cto[:, 64:]
        c128_r[0, :, :] = jnp.concatenate([cfr[:, :64], cnm[:, :64]], axis=1)
        c128_r[1, :, :] = jnp.concatenate([cfr[:, 64:], cnm[:, 64:]], axis=1)

    return pl.pallas_call(
        body,
        grid=(E_PAD // _EB,),
        in_specs=[pl.BlockSpec((_EB, 8), lambda i: (i, 0)),
                  _full((8, 128)), _full((8, 128)), _full((8, 128))],
        out_specs=[pl.BlockSpec((2, _EB, 64), lambda i: (0, i, 0)),
                   pl.BlockSpec((2, _EB, 128), lambda i: (0, i, 0))],
        out_shape=[jax.ShapeDtypeStruct((2, E_PAD, 64), _f32),
                   jax.ShapeDtypeStruct((2, E_PAD, 128), _f32)],
    )(ea8, wto, wfr, wnm)


def _proj_out_specs():
    return [pl.BlockSpec((2, _NB, 64), lambda i: (0, i, 0)),
            pl.BlockSpec((2, _NB, 64), lambda i: (0, i, 0)),
            pl.BlockSpec((2, _NB, 128), lambda i: (0, i, 0)),
            pl.BlockSpec((2, _NB, 128), lambda i: (0, i, 0))]


def _proj_out_shapes():
    return [jax.ShapeDtypeStruct((2, N, 64), _f32),
            jax.ShapeDtypeStruct((2, N, 64), _f32),
            jax.ShapeDtypeStruct((2, N, 128), _f32),
            jax.ShapeDtypeStruct((2, N, 128), _f32)]


def _write_proj(h, wproj_r, toA_r, toB_r, fnA_r, fnB_r):
    proj = jnp.dot(h, wproj_r[...], preferred_element_type=_f32)
    toA_r[0, :, :] = proj[:, 0:64]
    toA_r[1, :, :] = proj[:, 64:128]
    toB_r[0, :, :] = proj[:, 128:192]
    toB_r[1, :, :] = proj[:, 192:256]
    fnA_r[0, :, :] = proj[:, 256:384]
    fnA_r[1, :, :] = proj[:, 384:512]
    fnB_r[0, :, :] = proj[:, 512:640]
    fnB_r[1, :, :] = proj[:, 640:768]


def _tc_encoder(xp, encv, encW2, wproj):
    def body(xp_r, encv_r, encW2_r, wproj_r, h_r, toA_r, toB_r, fnA_r, fnB_r):
        x0 = xp_r[:, 0:1]
        h1 = jax.nn.relu(x0 * encv_r[0:1, :] + encv_r[1:2, :])
        h = jnp.dot(h1, encW2_r[...], preferred_element_type=_f32) + encv_r[2:3, :]
        h_r[...] = h
        _write_proj(h, wproj_r, toA_r, toB_r, fnA_r, fnB_r)

    return pl.pallas_call(
        body,
        grid=(N // _NB,),
        in_specs=[pl.BlockSpec((_NB, 128), lambda i: (i, 0)),
                  _full((8, 128)), _full((128, 128)), _full((128, 768))],
        out_specs=[pl.BlockSpec((_NB, 128), lambda i: (i, 0))] + _proj_out_specs(),
        out_shape=[jax.ShapeDtypeStruct((N, 128), _f32)] + _proj_out_shapes(),
    )(xp, encv, encW2, wproj)


def _tc_update(h, h_init, s_to, s_fn, extras, W):
    def body(h_r, hi_r, sto_r, sfn_r, ex_r,
             toW2_r, frW2_r, nmW2_r, upW1h_r, upW1t_r, upW1f_r, upx_r, upW2_r,
             unW1h_r, unW1n_r, unx_r, unW2_r, alpack_r, vecs_r, wproj_r,
             hn_r, toA_r, toB_r, fnA_r, fnB_r):
        hb = h_r[...]
        ex = ex_r[...]
        dot = lambda a, b: jnp.dot(a, b, preferred_element_type=_f32)
        mp_to = (dot(sto_r[0, :, :], toW2_r[0:64, :]) +
                 dot(sto_r[1, :, :], toW2_r[64:128, :]) + ex[:, 8:9] * vecs_r[0:1, :])
        mp_fr = (dot(sfn_r[0, :, 0:64], frW2_r[0:64, :]) +
                 dot(sfn_r[1, :, 0:64], frW2_r[64:128, :]) + ex[:, 9:10] * vecs_r[1:2, :])
        mp_nm = (dot(sfn_r[0, :, 64:128], nmW2_r[0:64, :]) +
                 dot(sfn_r[1, :, 64:128], nmW2_r[64:128, :]) + ex[:, 9:10] * vecs_r[2:3, :])

        logit = jnp.sum(hb * alpack_r[0:1, :] + mp_to * alpack_r[1:2, :] +
                        mp_fr * alpack_r[2:3, :] + ex * alpack_r[3:4, :],
                        axis=1, keepdims=True)
        alpha = jax.nn.sigmoid(logit)
        u1 = jax.nn.relu(dot(hb, upW1h_r[...]) + dot(mp_to, upW1t_r[...]) +
                         dot(mp_fr, upW1f_r[...]) + dot(ex, upx_r[...]))
        upd_int = alpha * (dot(u1, upW2_r[...]) + vecs_r[3:4, :])
        n1 = jax.nn.relu(dot(hb, unW1h_r[...]) + dot(mp_nm, unW1n_r[...]) +
                         dot(ex, unx_r[...]))
        upd_neu = dot(n1, unW2_r[...]) + vecs_r[4:5, :]

        hn = hb + upd_int
        hn = jnp.where(ex[:, 7:8] > 0.5, upd_neu, hn)
        mu = jnp.mean(hn, axis=1, keepdims=True)
        var = jnp.mean((hn - mu) * (hn - mu), axis=1, keepdims=True)
        hn = (hn - mu) * lax.rsqrt(var + 1e-5) * vecs_r[5:6, :] + vecs_r[6:7, :]
        hn = jnp.where(ex[:, 6:7] > 0.5, hi_r[...], hn)
        hn_r[...] = hn
        _write_proj(hn, wproj_r, toA_r, toB_r, fnA_r, fnB_r)

    nb = pl.BlockSpec((_NB, 128), lambda i: (i, 0))
    return pl.pallas_call(
        body,
        grid=(N // _NB,),
        in_specs=[nb, nb,
                  pl.BlockSpec((2, _NB, 64), lambda i: (0, i, 0)),
                  pl.BlockSpec((2, _NB, 128), lambda i: (0, i, 0)),
                  nb,
                  _full((128, 128)), _full((128, 128)), _full((128, 128)),
                  _full((128, 128)), _full((128, 128)), _full((128, 128)),
                  _full((128, 128)), _full((128, 128)), _full((128, 128)),
                  _full((128, 128)), _full((128, 128)), _full((128, 128)),
                  _full((8, 128)), _full((8, 128)), _full((128, 768))],
        out_specs=[nb] + _proj_out_specs(),
        out_shape=[jax.ShapeDtypeStruct((N, 128), _f32)] + _proj_out_shapes(),
    )(h, h_init, s_to, s_fn, extras, *W)


def _tc_decoder(h, decW1, decv):
    def body(h_r, decW1_r, decv_r, out_r):
        u1 = jax.nn.relu(jnp.dot(h_r[...], decW1_r[...],
                                 preferred_element_type=_f32) + decv_r[0:1, :])
        val = jnp.sum(u1 * decv_r[1:2, :], axis=1, keepdims=True) + decv_r[2:3, 0:1]
        out_r[...] = jnp.broadcast_to(val, (_NB, 128))

    return pl.pallas_call(
        body,
        grid=(N // _NB,),
        in_specs=[pl.BlockSpec((_NB, 128), lambda i: (i, 0)),
                  _full((128, 128)), _full((8, 128))],
        out_specs=pl.BlockSpec((_NB, 128), lambda i: (i, 0)),
        out_shape=jax.ShapeDtypeStruct((N, 128), _f32),
    )(h, decW1, decv)


def kernel(x, edge_index, edge_attr, prb_data, unit_normal_vector, tags, params):
    p = params
    src = edge_index[0]
    dst = edge_index[1]
    keep = src != dst
    dst_m = jnp.where(keep, dst, N)
    src_m = jnp.where(keep, src, N)

    pad = E_PAD - E
    dst_p = jnp.pad(dst, (0, pad)).astype(jnp.int32)
    src_p = jnp.pad(src, (0, pad)).astype(jnp.int32)
    dstm_p = jnp.pad(dst_m, (0, pad), constant_values=N).astype(jnp.int32)
    srcm_p = jnp.pad(src_m, (0, pad), constant_values=N).astype(jnp.int32)
    gidx = jnp.stack([dst_p, src_p, dstm_p, srcm_p])
    g3to = jnp.stack([dst_p, src_p, dstm_p]).reshape(3, E_PAD // 128, 128).transpose(1, 0, 2)
    g3fn = jnp.stack([src_p, dst_p, srcm_p]).reshape(3, E_PAD // 64, 64).transpose(1, 0, 2)

    ea8 = jnp.zeros((E_PAD, 8), _f32)
    ea8 = ea8.at[:E, 0:3].set(edge_attr)
    ea8 = ea8.at[:E, 3].set(1.0)

    def wpack(W1, b1):
        w = jnp.zeros((8, 128), _f32)
        return w.at[0:3, :].set(W1[256:259]).at[3, :].set(b1)

    wto = wpack(p['to_W1'], p['to_b1'])
    wfr = wpack(p['fr_W1'], p['fr_b1'])
    wnm = wpack(p['nm_W1'], p['nm_b1'])

    toA = p['to_W1'][:128]
    toB = p['to_W1'][128:256]
    frA, frB = p['fr_W1'][:128], p['fr_W1'][128:256]
    nmA, nmB = p['nm_W1'][:128], p['nm_W1'][128:256]
    wproj = jnp.concatenate([
        toA[:, :64], toA[:, 64:], toB[:, :64], toB[:, 64:],
        frA[:, :64], nmA[:, :64], frA[:, 64:], nmA[:, 64:],
        frB[:, :64], nmB[:, :64], frB[:, 64:], nmB[:, 64:],
    ], axis=1)

    encv = jnp.zeros((8, 128), _f32)
    encv = encv.at[0].set(p['enc_W1'][0]).at[1].set(p['enc_b1']).at[2].set(p['enc_b2'])
    decv = jnp.zeros((8, 128), _f32)
    decv = decv.at[0].set(p['dec_b1']).at[1].set(p['dec_W2'][:, 0]).at[2, 0].set(p['dec_b2'][0])

    alpack = jnp.zeros((8, 128), _f32)
    alpack = (alpack.at[0].set(p['al_W'][0:128, 0])
              .at[1].set(p['al_W'][128:256, 0])
              .at[2].set(p['al_W'][256:384, 0])
              .at[3, 0:3].set(p['al_W'][384:387, 0])
              .at[3, 5].set(p['al_b'][0]))
    upx = jnp.zeros((128, 128), _f32)
    upx = upx.at[0:3].set(p['up_W1'][384:387]).at[5].set(p['up_b1'])
    unx = jnp.zeros((128, 128), _f32)
    unx = (unx.at[0:3].set(p['un_W1'][256:259])
           .at[3:5].set(p['un_W1'][259:261]).at[5].set(p['un_b1']))
    vecs = jnp.stack([p['to_b2'], p['fr_b2'], p['nm_b2'], p['up_b2'],
                      p['un_b2'], p['ln_g'], p['ln_b'], jnp.zeros((128,), _f32)])

    W = (p['to_W2'], p['fr_W2'], p['nm_W2'],
         p['up_W1'][0:128], p['up_W1'][128:256], p['up_W1'][256:384], upx, p['up_W2'],
         p['un_W1'][0:128], p['un_W1'][128:256], unx, p['un_W2'],
         alpack, vecs, wproj)

    c64, c128 = _tc_edge_consts(ea8, wto, wfr, wnm)
    degs = _sc_degrees(gidx)

    extras = jnp.zeros((N, 128), _f32)
    extras = (extras.at[:, 0:3].set(prb_data)
              .at[:, 3:5].set(unit_normal_vector)
              .at[:, 5].set(1.0)
              .at[:, 6].set((tags[:, 1] == 1).astype(_f32))
              .at[:, 7].set((tags[:, 2] == 1).astype(_f32))
              .at[:, 8].set(degs[0, :N, 0])
              .at[:, 9].set(degs[1, :N, 0]))

    xp = jnp.zeros((N, 128), _f32).at[:, 0:1].set(x)
    h_init, toAt, toBt, fnAt, fnBt = _tc_encoder(xp, encv, p['enc_W2'], wproj)

    h = h_init
    for _ in range(ITERS):
        s_to = _sc_phi(toAt, toBt, c64, g3to, 64, 128, 80)
        s_fn = _sc_phi(fnAt, fnBt, c128, g3fn, 128, 64, 160)
        h, toAt, toBt, fnAt, fnBt = _tc_update(h, h_init, s_to, s_fn, extras, W)

    out = _tc_decoder(h, p['dec_W1'], decv)
    return out[:, 0:1]


# speedup vs baseline: 1.9908x; 1.1363x over previous
import functools

import jax
import jax.numpy as jnp
from jax import lax
from jax.experimental import pallas as pl
from jax.experimental.pallas import tpu as pltpu
from jax.experimental.pallas import tpu_sc as plsc

N = 10000
E = 160000
L = 128
ITERS = 2

NC = 2
NS = 16
CH = 128
NCH = 80
EPS = CH * NCH
E_PAD = EPS * NS
NT = E_PAD // CH
NROWS = 10112
ZPS = NROWS // NS
NRW = 10008
OB = 624
OW = 640

_f32 = jnp.float32


def _sc_phi(tA, tB, cc, gidx, ia, ib, isc, W, ch, nch):
    mesh = plsc.VectorSubcoreMesh(core_axis_name="c", subcore_axis_name="s",
                                  num_cores=NC, num_subcores=NS)
    nv = W // 16
    assert (nch - 4) % 4 == 0 and ch * nch == EPS

    @functools.partial(
        pl.kernel,
        out_type=jax.ShapeDtypeStruct((2, NRW, W), _f32),
        mesh=mesh,
        compiler_params=pltpu.CompilerParams(use_tc_tiling_on_sc=False),
        scratch_types=[
            pltpu.VMEM((ch, W), _f32), pltpu.VMEM((ch, W), _f32),
            pltpu.VMEM((ch, W), _f32),
            pltpu.VMEM((ch, W), _f32), pltpu.VMEM((ch, W), _f32),
            pltpu.VMEM((ch, W), _f32),
            pltpu.VMEM((3, ch), jnp.int32), pltpu.VMEM((3, ch), jnp.int32),
            pltpu.VMEM((3, ch), jnp.int32), pltpu.VMEM((3, ch), jnp.int32),
            pltpu.VMEM_SHARED((NRW, W), _f32),
            pltpu.SemaphoreType.DMA, pltpu.SemaphoreType.DMA,
            pltpu.SemaphoreType.DMA, pltpu.SemaphoreType.DMA,
            pltpu.SemaphoreType.DMA, pltpu.SemaphoreType.DMA,
            pltpu.SemaphoreType.DMA, pltpu.SemaphoreType.DMA,
        ],
    )
    def body(tAr, tBr, ccr, gidxr, outr,
             bufA0, bufB0, bufC0, bufA1, bufB1, bufC1,
             ix0, ix1, ix2, ix3, acc,
             semG0, semG1, semS0, semS1, semI0, semI1, semI2, semI3):
        cid = lax.axis_index("c")
        sid = lax.axis_index("s")
        data = ((bufA0, bufB0, bufC0, semG0, semS0),
                (bufA1, bufB1, bufC1, semG1, semS1))
        ixs = ((ix0, semI0), (ix1, semI1), (ix2, semI2), (ix3, semI3))

        @pl.loop(0, ch)
        def _(i):
            for j in range(nv):
                bufA0[i, pl.ds(j * 16, 16)] = jnp.zeros((16,), _f32)

        obase = sid * OB
        for k in range(OW // ch):
            pltpu.sync_copy(bufA0.at[pl.ds(0, ch)],
                            acc.at[pl.ds(obase + k * ch, ch)])
        plsc.subcore_barrier()

        def fire_idx(s, t):
            ix, semI = ixs[s]
            pltpu.async_copy(gidxr.at[ia].at[pl.ds(t * ch, ch)], ix.at[0], semI)
            pltpu.async_copy(gidxr.at[ib].at[pl.ds(t * ch, ch)], ix.at[1], semI)
            pltpu.async_copy(gidxr.at[isc].at[pl.ds(t * ch, ch)], ix.at[2], semI)

        def wI(s):
            ix, semI = ixs[s]
            for r in range(3):
                pltpu.make_async_copy(gidxr.at[0].at[pl.ds(0, ch)], ix.at[r], semI).wait()

        def fire_gathers(p, s, t):
            bA, bB, bC, semG, semS = data[p]
            ix, _ = ixs[s]
            pltpu.async_copy(tAr.at[cid].at[ix.at[0]], bA, semG)
            pltpu.async_copy(tBr.at[cid].at[ix.at[1]], bB, semG)
            pltpu.async_copy(ccr.at[cid].at[pl.ds(t * ch, ch)], bC, semG)

        def wG(p):
            bA, bB, bC, semG, semS = data[p]
            pltpu.make_async_copy(tAr.at[cid].at[ixs[0][0].at[0]], bA, semG).wait()
            pltpu.make_async_copy(tAr.at[cid].at[ixs[0][0].at[0]], bB, semG).wait()
            pltpu.make_async_copy(ccr.at[cid].at[pl.ds(0, ch)], bC, semG).wait()

        def relu_combine(p):
            bA, bB, bC, semG, semS = data[p]

            @plsc.parallel_loop(0, ch, unroll=2)
            def _(i):
                for j in range(nv):
                    sl = pl.ds(j * 16, 16)
                    bA[i, sl] = jnp.maximum(bA[i, sl] + bB[i, sl] + bC[i, sl], 0.0)

        def fire_scatter(p, s):
            bA, bB, bC, semG, semS = data[p]
            pltpu.async_copy(bA, acc.at[ixs[s][0].at[2]], semS, add=True)

        def wS(p):
            bA, bB, bC, semG, semS = data[p]
            pltpu.make_async_copy(bA, acc.at[ixs[0][0].at[2]], semS).wait()

        def step(p, s, t, *, do_ws=True, nxt=None, last=False):
            wG(p)
            if do_ws:
                wS(1 - p)
            if nxt is not None:
                fire_idx(nxt[0], nxt[1])
            if not last:
                wI((s + 1) % 4)
                fire_gathers(1 - p, (s + 1) % 4, t + 1)
            relu_combine(p)
            fire_scatter(p, s)

        T = sid * nch
        fire_idx(0, T)
        fire_idx(1, T + 1)
        wI(0)
        fire_gathers(0, 0, T)
        step(0, 0, T, do_ws=False, nxt=(2, T + 2))
        step(1, 1, T + 1, nxt=(3, T + 3))

        @pl.loop(0, (nch - 4) // 4)
        def _(j):
            c = T + 2 + 4 * j
            step(0, 2, c, nxt=(0, c + 2))
            step(1, 3, c + 1, nxt=(1, c + 3))
            step(0, 0, c + 2, nxt=(2, c + 4))
            step(1, 1, c + 3, nxt=(3, c + 5))

        step(0, 2, T + nch - 2)
        step(1, 3, T + nch - 1, last=True)
        wS(1)

        plsc.subcore_barrier()

        for k in range(OW // ch):
            r = obase + k * ch
            pltpu.sync_copy(acc.at[pl.ds(r, ch)], outr.at[cid].at[pl.ds(r, ch)])

    return body(tA, tB, cc, gidx)


def _sc_degrees(gidx):
    mesh = plsc.VectorSubcoreMesh(core_axis_name="c", subcore_axis_name="s",
                                  num_cores=NC, num_subcores=NS)

    @functools.partial(
        pl.kernel,
        out_type=jax.ShapeDtypeStruct((2, NROWS, 16), _f32),
        mesh=mesh,
        compiler_params=pltpu.CompilerParams(use_tc_tiling_on_sc=False),
        scratch_types=[
            pltpu.VMEM((CH, 16), _f32),
            pltpu.VMEM((CH, 16), _f32),
            pltpu.VMEM((CH,), jnp.int32),
            pltpu.VMEM_SHARED((NROWS, 16), _f32),
        ],
    )
    def body(gidxr, degr, ones, zer, idx, acc):
        cid = lax.axis_index("c")
        sid = lax.axis_index("s")

        @pl.loop(0, CH)
        def _(i):
            ones[i, pl.ds(0, 16)] = jnp.ones((16,), _f32)
            zer[i, pl.ds(0, 16)] = jnp.zeros((16,), _f32)

        zbase = sid * ZPS
        for k in range(4):
            pltpu.sync_copy(zer.at[pl.ds(0, CH)], acc.at[pl.ds(zbase + k * CH, CH)])
        pltpu.sync_copy(zer.at[pl.ds(0, ZPS - 4 * CH)],
                        acc.at[pl.ds(zbase + 4 * CH, ZPS - 4 * CH)])
        plsc.subcore_barrier()

        @pl.loop(0, NCH)
        def _(g):
            base = sid * EPS + g * CH

            @pl.when(cid == 0)
            def _():
                pltpu.sync_copy(gidxr.at[2].at[pl.ds(base, CH)], idx)

            @pl.when(cid == 1)
            def _():
                pltpu.sync_copy(gidxr.at[3].at[pl.ds(base, CH)], idx)

            pltpu.sync_copy(ones, acc.at[idx], add=True)

        plsc.subcore_barrier()

        def copy_out(c):
            for k in range(5):
                nrow = CH if k < 4 else ZPS - 4 * CH
                r = zbase + k * CH
                pltpu.sync_copy(acc.at[pl.ds(r, nrow)], degr.at[c].at[pl.ds(r, nrow)])

        @pl.when(cid == 0)
        def _():
            copy_out(0)

        @pl.when(cid == 1)
        def _():
            copy_out(1)

    return body(gidx)


_NB = 2000
_EB = 1000


def _full(shape):
    return pl.BlockSpec(shape, lambda i: tuple(0 for _ in shape))


def _tc_edge_consts(edge_attr, wto, wfr, wnm):
    def body(ea_r, wto_r, wfr_r, wnm_r, c64_r, c128_r):
        ea = ea_r[...]
        cto = jnp.dot(ea, wto_r[0:3, :], preferred_element_type=_f32) + wto_r[3:4, :]
        cfr = jnp.dot(ea, wfr_r[0:3, :], preferred_element_type=_f32) + wfr_r[3:4, :]
        cnm = jnp.dot(ea, wnm_r[0:3, :], preferred_element_type=_f32) + wnm_r[3:4, :]
        c64_r[0, :, :] = cto[:, :64]
        c64_r[1, :, :] = cto[:, 64:]
        c128_r[0, :, :] = jnp.concatenate([cfr[:, :64], cnm[:, :64]], axis=1)
        c128_r[1, :, :] = jnp.concatenate([cfr[:, 64:], cnm[:, 64:]], axis=1)

    return pl.pallas_call(
        body,
        grid=(E // _EB,),
        in_specs=[pl.BlockSpec((_EB, 3), lambda i: (i, 0)),
                  _full((8, 128)), _full((8, 128)), _full((8, 128))],
        out_specs=[pl.BlockSpec((2, _EB, 64), lambda i: (0, i, 0)),
                   pl.BlockSpec((2, _EB, 128), lambda i: (0, i, 0))],
        out_shape=[jax.ShapeDtypeStruct((2, E_PAD, 64), _f32),
                   jax.ShapeDtypeStruct((2, E_PAD, 128), _f32)],
    )(edge_attr, wto, wfr, wnm)


def _proj_out_specs():
    return [pl.BlockSpec((2, _NB, 64), lambda i: (0, i, 0)),
            pl.BlockSpec((2, _NB, 64), lambda i: (0, i, 0)),
            pl.BlockSpec((2, _NB, 128), lambda i: (0, i, 0)),
            pl.BlockSpec((2, _NB, 128), lambda i: (0, i, 0))]


def _proj_out_shapes():
    return [jax.ShapeDtypeStruct((2, N, 64), _f32),
            jax.ShapeDtypeStruct((2, N, 64), _f32),
            jax.ShapeDtypeStruct((2, N, 128), _f32),
            jax.ShapeDtypeStruct((2, N, 128), _f32)]


def _write_proj(h, wproj_r, toA_r, toB_r, fnA_r, fnB_r):
    proj = jnp.dot(h, wproj_r[...], preferred_element_type=_f32)
    toA_r[0, :, :] = proj[:, 0:64]
    toA_r[1, :, :] = proj[:, 64:128]
    toB_r[0, :, :] = proj[:, 128:192]
    toB_r[1, :, :] = proj[:, 192:256]
    fnA_r[0, :, :] = proj[:, 256:384]
    fnA_r[1, :, :] = proj[:, 384:512]
    fnB_r[0, :, :] = proj[:, 512:640]
    fnB_r[1, :, :] = proj[:, 640:768]


def _tc_encoder(xp, encv, encW2, wproj):
    def body(xp_r, encv_r, encW2_r, wproj_r, h_r, toA_r, toB_r, fnA_r, fnB_r):
        x0 = xp_r[:, 0:1]
        h1 = jax.nn.relu(x0 * encv_r[0:1, :] + encv_r[1:2, :])
        h = jnp.dot(h1, encW2_r[...], preferred_element_type=_f32) + encv_r[2:3, :]
        h_r[...] = h
        _write_proj(h, wproj_r, toA_r, toB_r, fnA_r, fnB_r)

    return pl.pallas_call(
        body,
        grid=(N // _NB,),
        in_specs=[pl.BlockSpec((_NB, 128), lambda i: (i, 0)),
                  _full((8, 128)), _full((128, 128)), _full((128, 768))],
        out_specs=[pl.BlockSpec((_NB, 128), lambda i: (i, 0))] + _proj_out_specs(),
        out_shape=[jax.ShapeDtypeStruct((N, 128), _f32)] + _proj_out_shapes(),
    )(xp, encv, encW2, wproj)


def _tc_update(h, h_init, s_to, s_fn, prb, unv, tags, degs, W):
    def body(h_r, hi_r, sto_r, sfn_r, prb_r, unv_r, tags_r, degs_r,
             toW2_r, frW2_r, nmW2_r, upW1h_r, upW1t_r, upW1f_r, upW2_r,
             unW1h_r, unW1n_r, unW2_r, vecs_r, wext_r, wproj_r,
             hn_r, toA_r, toB_r, fnA_r, fnB_r):
        hb = h_r[...]
        prb = prb_r[...]
        unv = unv_r[...]
        tg = tags_r[...]
        deg_dst = degs_r[0, :, 0:1]
        deg_src = degs_r[1, :, 0:1]
        dot = lambda a, b: jnp.dot(a, b, preferred_element_type=_f32)
        mp_to = (dot(sto_r[0, :, :], toW2_r[0:64, :]) +
                 dot(sto_r[1, :, :], toW2_r[64:128, :]) + deg_dst * vecs_r[0:1, :])
        mp_fr = (dot(sfn_r[0, :, 0:64], frW2_r[0:64, :]) +
                 dot(sfn_r[1, :, 0:64], frW2_r[64:128, :]) + deg_src * vecs_r[1:2, :])
        mp_nm = (dot(sfn_r[0, :, 64:128], nmW2_r[0:64, :]) +
                 dot(sfn_r[1, :, 64:128], nmW2_r[64:128, :]) + deg_src * vecs_r[2:3, :])

        logit = (jnp.sum(hb * vecs_r[9:10, :] + mp_to * vecs_r[10:11, :] +
                         mp_fr * vecs_r[11:12, :], axis=1, keepdims=True) +
                 prb[:, 0:1] * vecs_r[12:13, 0:1] +
                 prb[:, 1:2] * vecs_r[12:13, 1:2] +
                 prb[:, 2:3] * vecs_r[12:13, 2:3] + vecs_r[12:13, 3:4])
        alpha = jax.nn.sigmoid(logit)
        u1 = jax.nn.relu(dot(hb, upW1h_r[...]) + dot(mp_to, upW1t_r[...]) +
                         dot(mp_fr, upW1f_r[...]) + dot(prb, wext_r[0:3, :]) +
                         vecs_r[3:4, :])
        upd_int = alpha * (dot(u1, upW2_r[...]) + vecs_r[4:5, :])
        n1 = jax.nn.relu(dot(hb, unW1h_r[...]) + dot(mp_nm, unW1n_r[...]) +
                         dot(prb, wext_r[3:6, :]) + dot(unv, wext_r[6:8, :]) +
                         vecs_r[5:6, :])
        upd_neu = dot(n1, unW2_r[...]) + vecs_r[6:7, :]

        hn = hb + upd_int
        hn = jnp.where(tg[:, 2:3] == 1, upd_neu, hn)
        mu = jnp.mean(hn, axis=1, keepdims=True)
        var = jnp.mean((hn - mu) * (hn - mu), axis=1, keepdims=True)
        hn = (hn - mu) * lax.rsqrt(var + 1e-5) * vecs_r[7:8, :] + vecs_r[8:9, :]
        hn = jnp.where(tg[:, 1:2] == 1, hi_r[...], hn)
        hn_r[...] = hn
        _write_proj(hn, wproj_r, toA_r, toB_r, fnA_r, fnB_r)

    nb = pl.BlockSpec((_NB, 128), lambda i: (i, 0))
    return pl.pallas_call(
        body,
        grid=(N // _NB,),
        in_specs=[nb, nb,
                  pl.BlockSpec((2, _NB, 64), lambda i: (0, i, 0)),
                  pl.BlockSpec((2, _NB, 128), lambda i: (0, i, 0)),
                  pl.BlockSpec((_NB, 3), lambda i: (i, 0)),
                  pl.BlockSpec((_NB, 2), lambda i: (i, 0)),
                  pl.BlockSpec((_NB, 3), lambda i: (i, 0)),
                  pl.BlockSpec((2, _NB, 16), lambda i: (0, i, 0)),
                  _full((128, 128)), _full((128, 128)), _full((128, 128)),
                  _full((128, 128)), _full((128, 128)), _full((128, 128)),
                  _full((128, 128)), _full((128, 128)), _full((128, 128)),
                  _full((128, 128)),
                  _full((16, 128)), _full((8, 128)), _full((128, 768))],
        out_specs=[nb] + _proj_out_specs(),
        out_shape=[jax.ShapeDtypeStruct((N, 128), _f32)] + _proj_out_shapes(),
    )(h, h_init, s_to, s_fn, prb, unv, tags, degs, *W)


def _tc_decoder(h, decW1, decv):
    def body(h_r, decW1_r, decv_r, out_r):
        u1 = jax.nn.relu(jnp.dot(h_r[...], decW1_r[...],
                                 preferred_element_type=_f32) + decv_r[0:1, :])
        val = jnp.sum(u1 * decv_r[1:2, :], axis=1, keepdims=True) + decv_r[2:3, 0:1]
        out_r[...] = jnp.broadcast_to(val, (_NB, 128))

    return pl.pallas_call(
        body,
        grid=(N // _NB,),
        in_specs=[pl.BlockSpec((_NB, 128), lambda i: (i, 0)),
                  _full((128, 128)), _full((8, 128))],
        out_specs=pl.BlockSpec((_NB, 128), lambda i: (i, 0)),
        out_shape=jax.ShapeDtypeStruct((N, 128), _f32),
    )(h, decW1, decv)


def kernel(x, edge_index, edge_attr, prb_data, unit_normal_vector, tags, params):
    p = params
    src = edge_index[0]
    dst = edge_index[1]
    keep = src != dst
    dst_m = jnp.where(keep, dst, N)
    src_m = jnp.where(keep, src, N)

    pad = E_PAD - E
    gidx = jnp.stack([
        jnp.pad(dst, (0, pad)),
        jnp.pad(src, (0, pad)),
        jnp.pad(dst_m, (0, pad), constant_values=N),
        jnp.pad(src_m, (0, pad), constant_values=N),
    ]).astype(jnp.int32)

    def wpack(W1, b1):
        w = jnp.zeros((8, 128), _f32)
        return w.at[0:3, :].set(W1[256:259]).at[3, :].set(b1)

    wto = wpack(p['to_W1'], p['to_b1'])
    wfr = wpack(p['fr_W1'], p['fr_b1'])
    wnm = wpack(p['nm_W1'], p['nm_b1'])

    toA = p['to_W1'][:128]
    toB = p['to_W1'][128:256]
    frA, frB = p['fr_W1'][:128], p['fr_W1'][128:256]
    nmA, nmB = p['nm_W1'][:128], p['nm_W1'][128:256]
    wproj = jnp.concatenate([
        toA[:, :64], toA[:, 64:], toB[:, :64], toB[:, 64:],
        frA[:, :64], nmA[:, :64], frA[:, 64:], nmA[:, 64:],
        frB[:, :64], nmB[:, :64], frB[:, 64:], nmB[:, 64:],
    ], axis=1)

    encv = jnp.zeros((8, 128), _f32)
    encv = encv.at[0].set(p['enc_W1'][0]).at[1].set(p['enc_b1']).at[2].set(p['enc_b2'])
    decv = jnp.zeros((8, 128), _f32)
    decv = decv.at[0].set(p['dec_b1']).at[1].set(p['dec_W2'][:, 0]).at[2, 0].set(p['dec_b2'][0])

    al_misc = jnp.concatenate([p['al_W'][384:387, 0], p['al_b'],
                               jnp.zeros((124,), _f32)])
    vecs = jnp.stack([p['to_b2'], p['fr_b2'], p['nm_b2'], p['up_b1'],
                      p['up_b2'], p['un_b1'], p['un_b2'], p['ln_g'], p['ln_b'],
                      p['al_W'][0:128, 0], p['al_W'][128:256, 0],
                      p['al_W'][256:384, 0], al_misc,
                      jnp.zeros((128,), _f32), jnp.zeros((128,), _f32),
                      jnp.zeros((128,), _f32)])
    wext = jnp.concatenate([p['up_W1'][384:387], p['un_W1'][256:259],
                            p['un_W1'][259:261]], axis=0)

    W = (p['to_W2'], p['fr_W2'], p['nm_W2'],
         p['up_W1'][0:128], p['up_W1'][128:256], p['up_W1'][256:384], p['up_W2'],
         p['un_W1'][0:128], p['un_W1'][128:256], p['un_W2'],
         vecs, wext, wproj)

    c64, c128 = _tc_edge_consts(edge_attr, wto, wfr, wnm)
    degs = _sc_degrees(gidx)

    xp = jnp.zeros((N, 128), _f32).at[:, 0:1].set(x)
    h_init, toAt, toBt, fnAt, fnBt = _tc_encoder(xp, encv, p['enc_W2'], wproj)

    h = h_init
    for _ in range(ITERS):
        s_to = _sc_phi(toAt, toBt, c64, gidx, 0, 1, 2, 64, 128, 80)
        s_fn = _sc_phi(fnAt, fnBt, c128, gidx, 1, 0, 3, 128, 64, 160)
        h, toAt, toBt, fnAt, fnBt = _tc_update(
            h, h_init, s_to, s_fn, prb_data, unit_normal_vector, tags, degs, W)

    out = _tc_decoder(h, p['dec_W1'], decv)
    return out[:, 0:1]


# speedup vs baseline: 2.0179x; 1.0136x over previous
import functools

import jax
import jax.numpy as jnp
from jax import lax
from jax.experimental import pallas as pl
from jax.experimental.pallas import tpu as pltpu
from jax.experimental.pallas import tpu_sc as plsc

N = 10000
E = 160000
L = 128
ITERS = 2

NC = 2
NS = 16
CH = 128
NCH = 80
EPS = CH * NCH
E_PAD = EPS * NS
NT = E_PAD // CH
NROWS = 10112
ZPS = NROWS // NS
NRW = 10008
OB = 624
OW = 640

_f32 = jnp.float32


def _sc_phi(tA, tB, cc, iA, iB, iS, W, ch, nch):
    mesh = plsc.VectorSubcoreMesh(core_axis_name="c", subcore_axis_name="s",
                                  num_cores=NC, num_subcores=NS)
    nv = W // 16
    assert (nch - 4) % 4 == 0 and ch * nch == EPS

    @functools.partial(
        pl.kernel,
        out_type=jax.ShapeDtypeStruct((2, NRW, W), _f32),
        mesh=mesh,
        compiler_params=pltpu.CompilerParams(use_tc_tiling_on_sc=False),
        scratch_types=[
            pltpu.VMEM((ch, W), _f32), pltpu.VMEM((ch, W), _f32),
            pltpu.VMEM((ch, W), _f32),
            pltpu.VMEM((ch, W), _f32), pltpu.VMEM((ch, W), _f32),
            pltpu.VMEM((ch, W), _f32),
            pltpu.VMEM((3, ch), jnp.int32), pltpu.VMEM((3, ch), jnp.int32),
            pltpu.VMEM((3, ch), jnp.int32), pltpu.VMEM((3, ch), jnp.int32),
            pltpu.VMEM_SHARED((NRW, W), _f32),
            pltpu.SemaphoreType.DMA, pltpu.SemaphoreType.DMA,
            pltpu.SemaphoreType.DMA, pltpu.SemaphoreType.DMA,
            pltpu.SemaphoreType.DMA, pltpu.SemaphoreType.DMA,
            pltpu.SemaphoreType.DMA, pltpu.SemaphoreType.DMA,
        ],
    )
    def body(tAr, tBr, ccr, iAr, iBr, iSr, outr,
             bufA0, bufB0, bufC0, bufA1, bufB1, bufC1,
             ix0, ix1, ix2, ix3, acc,
             semG0, semG1, semS0, semS1, semI0, semI1, semI2, semI3):
        cid = lax.axis_index("c")
        sid = lax.axis_index("s")
        data = ((bufA0, bufB0, bufC0, semG0, semS0),
                (bufA1, bufB1, bufC1, semG1, semS1))
        ixs = ((ix0, semI0), (ix1, semI1), (ix2, semI2), (ix3, semI3))

        @pl.loop(0, ch)
        def _(i):
            for j in range(nv):
                bufA0[i, pl.ds(j * 16, 16)] = jnp.zeros((16,), _f32)

        obase = sid * OB
        for k in range(OW // ch):
            pltpu.sync_copy(bufA0.at[pl.ds(0, ch)],
                            acc.at[pl.ds(obase + k * ch, ch)])
        plsc.subcore_barrier()

        def fire_idx(s, t):
            ix, semI = ixs[s]
            pltpu.async_copy(iAr.at[pl.ds(t * ch, ch)], ix.at[0], semI)
            pltpu.async_copy(iBr.at[pl.ds(t * ch, ch)], ix.at[1], semI)
            pltpu.async_copy(iSr.at[pl.ds(t * ch, ch)], ix.at[2], semI)

        def wI(s):
            ix, semI = ixs[s]
            for r in range(3):
                pltpu.make_async_copy(iAr.at[pl.ds(0, ch)], ix.at[r], semI).wait()

        def fire_gathers(p, s, t):
            bA, bB, bC, semG, semS = data[p]
            ix, _ = ixs[s]
            pltpu.async_copy(tAr.at[cid].at[ix.at[0]], bA, semG)
            pltpu.async_copy(tBr.at[cid].at[ix.at[1]], bB, semG)
            pltpu.async_copy(ccr.at[cid].at[pl.ds(t * ch, ch)], bC, semG)

        def wG(p):
            bA, bB, bC, semG, semS = data[p]
            pltpu.make_async_copy(tAr.at[cid].at[ixs[0][0].at[0]], bA, semG).wait()
            pltpu.make_async_copy(tAr.at[cid].at[ixs[0][0].at[0]], bB, semG).wait()
            pltpu.make_async_copy(ccr.at[cid].at[pl.ds(0, ch)], bC, semG).wait()

        def relu_combine(p):
            bA, bB, bC, semG, semS = data[p]

            @plsc.parallel_loop(0, ch, unroll=2)
            def _(i):
                for j in range(nv):
                    sl = pl.ds(j * 16, 16)
                    bA[i, sl] = jnp.maximum(bA[i, sl] + bB[i, sl] + bC[i, sl], 0.0)

        def fire_scatter(p, s):
            bA, bB, bC, semG, semS = data[p]
            pltpu.async_copy(bA, acc.at[ixs[s][0].at[2]], semS, add=True)

        def wS(p):
            bA, bB, bC, semG, semS = data[p]
            pltpu.make_async_copy(bA, acc.at[ixs[0][0].at[2]], semS).wait()

        def step(p, s, t, *, do_ws=True, nxt=None, last=False):
            wG(p)
            if do_ws:
                wS(1 - p)
            if nxt is not None:
                fire_idx(nxt[0], nxt[1])
            if not last:
                wI((s + 1) % 4)
                fire_gathers(1 - p, (s + 1) % 4, t + 1)
            relu_combine(p)
            fire_scatter(p, s)

        T = sid * nch
        fire_idx(0, T)
        fire_idx(1, T + 1)
        wI(0)
        fire_gathers(0, 0, T)
        step(0, 0, T, do_ws=False, nxt=(2, T + 2))
        step(1, 1, T + 1, nxt=(3, T + 3))

        @pl.loop(0, (nch - 4) // 4)
        def _(j):
            c = T + 2 + 4 * j
            step(0, 2, c, nxt=(0, c + 2))
            step(1, 3, c + 1, nxt=(1, c + 3))
            step(0, 0, c + 2, nxt=(2, c + 4))
            step(1, 1, c + 3, nxt=(3, c + 5))

        step(0, 2, T + nch - 2)
        step(1, 3, T + nch - 1, last=True)
        wS(1)

        plsc.subcore_barrier()

        for k in range(OW // ch):
            r = obase + k * ch
            pltpu.sync_copy(acc.at[pl.ds(r, ch)], outr.at[cid].at[pl.ds(r, ch)])

    return body(tA, tB, cc, iA, iB, iS)


def _sc_degrees(dstm, srcm):
    mesh = plsc.VectorSubcoreMesh(core_axis_name="c", subcore_axis_name="s",
                                  num_cores=NC, num_subcores=NS)

    @functools.partial(
        pl.kernel,
        out_type=jax.ShapeDtypeStruct((2, NROWS, 16), _f32),
        mesh=mesh,
        compiler_params=pltpu.CompilerParams(use_tc_tiling_on_sc=False),
        scratch_types=[
            pltpu.VMEM((CH, 16), _f32),
            pltpu.VMEM((CH, 16), _f32),
            pltpu.VMEM((CH,), jnp.int32),
            pltpu.VMEM_SHARED((NROWS, 16), _f32),
        ],
    )
    def body(dmr, smr, degr, ones, zer, idx, acc):
        cid = lax.axis_index("c")
        sid = lax.axis_index("s")

        @pl.loop(0, CH)
        def _(i):
            ones[i, pl.ds(0, 16)] = jnp.ones((16,), _f32)
            zer[i, pl.ds(0, 16)] = jnp.zeros((16,), _f32)

        zbase = sid * ZPS
        for k in range(4):
            pltpu.sync_copy(zer.at[pl.ds(0, CH)], acc.at[pl.ds(zbase + k * CH, CH)])
        pltpu.sync_copy(zer.at[pl.ds(0, ZPS - 4 * CH)],
                        acc.at[pl.ds(zbase + 4 * CH, ZPS - 4 * CH)])
        plsc.subcore_barrier()

        @pl.loop(0, NCH)
        def _(g):
            base = sid * EPS + g * CH

            @pl.when(cid == 0)
            def _():
                pltpu.sync_copy(dmr.at[pl.ds(base, CH)], idx)

            @pl.when(cid == 1)
            def _():
                pltpu.sync_copy(smr.at[pl.ds(base, CH)], idx)

            pltpu.sync_copy(ones, acc.at[idx], add=True)

        plsc.subcore_barrier()

        def copy_out(c):
            for k in range(5):
                nrow = CH if k < 4 else ZPS - 4 * CH
                r = zbase + k * CH
                pltpu.sync_copy(acc.at[pl.ds(r, nrow)], degr.at[c].at[pl.ds(r, nrow)])

        @pl.when(cid == 0)
        def _():
            copy_out(0)

        @pl.when(cid == 1)
        def _():
            copy_out(1)

    return body(dstm, srcm)


_NB = 2000
_EB = 1000


def _full(shape):
    return pl.BlockSpec(shape, lambda i: tuple(0 for _ in shape))


def _tc_edge_consts(edge_attr, wto, wfr, wnm):
    def body64(ea_r, wto_r, c64_r):
        ea = ea_r[...]
        cto = jnp.dot(ea, wto_r[0:3, :], preferred_element_type=_f32) + wto_r[3:4, :]
        c64_r[0, :, :] = cto[:, :64]
        c64_r[1, :, :] = cto[:, 64:]

    def body128(ea_r, wfr_r, wnm_r, c128_r):
        ea = ea_r[...]
        cfr = jnp.dot(ea, wfr_r[0:3, :], preferred_element_type=_f32) + wfr_r[3:4, :]
        cnm = jnp.dot(ea, wnm_r[0:3, :], preferred_element_type=_f32) + wnm_r[3:4, :]
        c128_r[0, :, :] = jnp.concatenate([cfr[:, :64], cnm[:, :64]], axis=1)
        c128_r[1, :, :] = jnp.concatenate([cfr[:, 64:], cnm[:, 64:]], axis=1)

    c64 = pl.pallas_call(
        body64,
        grid=(E // _EB,),
        in_specs=[pl.BlockSpec((_EB, 3), lambda i: (i, 0)), _full((8, 128))],
        out_specs=pl.BlockSpec((2, _EB, 64), lambda i: (0, i, 0)),
        out_shape=jax.ShapeDtypeStruct((2, E_PAD, 64), _f32),
    )(edge_attr, wto)
    c128 = pl.pallas_call(
        body128,
        grid=(E // _EB,),
        in_specs=[pl.BlockSpec((_EB, 3), lambda i: (i, 0)),
                  _full((8, 128)), _full((8, 128))],
        out_specs=pl.BlockSpec((2, _EB, 128), lambda i: (0, i, 0)),
        out_shape=jax.ShapeDtypeStruct((2, E_PAD, 128), _f32),
    )(edge_attr, wfr, wnm)
    return c64, c128


def _proj_out_specs():
    return [pl.BlockSpec((2, _NB, 64), lambda i: (0, i, 0)),
            pl.BlockSpec((2, _NB, 64), lambda i: (0, i, 0)),
            pl.BlockSpec((2, _NB, 128), lambda i: (0, i, 0)),
            pl.BlockSpec((2, _NB, 128), lambda i: (0, i, 0))]


def _proj_out_shapes():
    return [jax.ShapeDtypeStruct((2, N, 64), _f32),
            jax.ShapeDtypeStruct((2, N, 64), _f32),
            jax.ShapeDtypeStruct((2, N, 128), _f32),
            jax.ShapeDtypeStruct((2, N, 128), _f32)]


def _write_proj(h, wproj_r, toA_r, toB_r, fnA_r, fnB_r):
    proj = jnp.dot(h, wproj_r[...], preferred_element_type=_f32)
    toA_r[0, :, :] = proj[:, 0:64]
    toA_r[1, :, :] = proj[:, 64:128]
    toB_r[0, :, :] = proj[:, 128:192]
    toB_r[1, :, :] = proj[:, 192:256]
    fnA_r[0, :, :] = proj[:, 256:384]
    fnA_r[1, :, :] = proj[:, 384:512]
    fnB_r[0, :, :] = proj[:, 512:640]
    fnB_r[1, :, :] = proj[:, 640:768]


def _tc_encoder(xp, encv, encW2, wproj):
    def body(xp_r, encv_r, encW2_r, wproj_r, h_r, toA_r, toB_r, fnA_r, fnB_r):
        x0 = xp_r[:, 0:1]
        h1 = jax.nn.relu(x0 * encv_r[0:1, :] + encv_r[1:2, :])
        h = jnp.dot(h1, encW2_r[...], preferred_element_type=_f32) + encv_r[2:3, :]
        h_r[...] = h
        _write_proj(h, wproj_r, toA_r, toB_r, fnA_r, fnB_r)

    return pl.pallas_call(
        body,
        grid=(N // _NB,),
        in_specs=[pl.BlockSpec((_NB, 128), lambda i: (i, 0)),
                  _full((8, 128)), _full((128, 128)), _full((128, 768))],
        out_specs=[pl.BlockSpec((_NB, 128), lambda i: (i, 0))] + _proj_out_specs(),
        out_shape=[jax.ShapeDtypeStruct((N, 128), _f32)] + _proj_out_shapes(),
    )(xp, encv, encW2, wproj)


def _tc_update(h, h_init, s_to, s_fn, prb, unv, tags, degs, W):
    def body(h_r, hi_r, sto_r, sfn_r, prb_r, unv_r, tags_r, degs_r,
             toW2_r, frW2_r, nmW2_r, upW1h_r, upW1t_r, upW1f_r, upW2_r,
             unW1h_r, unW1n_r, unW2_r, vecs_r, wext_r, wproj_r,
             hn_r, toA_r, toB_r, fnA_r, fnB_r):
        hb = h_r[...]
        prb = prb_r[...]
        unv = unv_r[...]
        tg = tags_r[...]
        deg_dst = degs_r[0, :, 0:1]
        deg_src = degs_r[1, :, 0:1]
        dot = lambda a, b: jnp.dot(a, b, preferred_element_type=_f32)
        mp_to = (dot(sto_r[0, :, :], toW2_r[0:64, :]) +
                 dot(sto_r[1, :, :], toW2_r[64:128, :]) + deg_dst * vecs_r[0:1, :])
        mp_fr = (dot(sfn_r[0, :, 0:64], frW2_r[0:64, :]) +
                 dot(sfn_r[1, :, 0:64], frW2_r[64:128, :]) + deg_src * vecs_r[1:2, :])
        mp_nm = (dot(sfn_r[0, :, 64:128], nmW2_r[0:64, :]) +
                 dot(sfn_r[1, :, 64:128], nmW2_r[64:128, :]) + deg_src * vecs_r[2:3, :])

        logit = (jnp.sum(hb * vecs_r[9:10, :] + mp_to * vecs_r[10:11, :] +
                         mp_fr * vecs_r[11:12, :], axis=1, keepdims=True) +
                 prb[:, 0:1] * vecs_r[12:13, 0:1] +
                 prb[:, 1:2] * vecs_r[12:13, 1:2] +
                 prb[:, 2:3] * vecs_r[12:13, 2:3] + vecs_r[12:13, 3:4])
        alpha = jax.nn.sigmoid(logit)
        u1 = jax.nn.relu(dot(hb, upW1h_r[...]) + dot(mp_to, upW1t_r[...]) +
                         dot(mp_fr, upW1f_r[...]) + dot(prb, wext_r[0:3, :]) +
                         vecs_r[3:4, :])
        upd_int = alpha * (dot(u1, upW2_r[...]) + vecs_r[4:5, :])
        n1 = jax.nn.relu(dot(hb, unW1h_r[...]) + dot(mp_nm, unW1n_r[...]) +
                         dot(prb, wext_r[3:6, :]) + dot(unv, wext_r[6:8, :]) +
                         vecs_r[5:6, :])
        upd_neu = dot(n1, unW2_r[...]) + vecs_r[6:7, :]

        hn = hb + upd_int
        hn = jnp.where(tg[:, 2:3] == 1, upd_neu, hn)
        mu = jnp.mean(hn, axis=1, keepdims=True)
        var = jnp.mean((hn - mu) * (hn - mu), axis=1, keepdims=True)
        hn = (hn - mu) * lax.rsqrt(var + 1e-5) * vecs_r[7:8, :] + vecs_r[8:9, :]
        hn = jnp.where(tg[:, 1:2] == 1, hi_r[...], hn)
        hn_r[...] = hn
        _write_proj(hn, wproj_r, toA_r, toB_r, fnA_r, fnB_r)

    nb = pl.BlockSpec((_NB, 128), lambda i: (i, 0))
    return pl.pallas_call(
        body,
        grid=(N // _NB,),
        in_specs=[nb, nb,
                  pl.BlockSpec((2, _NB, 64), lambda i: (0, i, 0)),
                  pl.BlockSpec((2, _NB, 128), lambda i: (0, i, 0)),
                  pl.BlockSpec((_NB, 3), lambda i: (i, 0)),
                  pl.BlockSpec((_NB, 2), lambda i: (i, 0)),
                  pl.BlockSpec((_NB, 3), lambda i: (i, 0)),
                  pl.BlockSpec((2, _NB, 16), lambda i: (0, i, 0)),
                  _full((128, 128)), _full((128, 128)), _full((128, 128)),
                  _full((128, 128)), _full((128, 128)), _full((128, 128)),
                  _full((128, 128)), _full((128, 128)), _full((128, 128)),
                  _full((128, 128)),
                  _full((16, 128)), _full((8, 128)), _full((128, 768))],
        out_specs=[nb] + _proj_out_specs(),
        out_shape=[jax.ShapeDtypeStruct((N, 128), _f32)] + _proj_out_shapes(),
    )(h, h_init, s_to, s_fn, prb, unv, tags, degs, *W)


def _tc_decoder(h, decW1, decv):
    def body(h_r, decW1_r, decv_r, out_r):
        u1 = jax.nn.relu(jnp.dot(h_r[...], decW1_r[...],
                                 preferred_element_type=_f32) + decv_r[0:1, :])
        val = jnp.sum(u1 * decv_r[1:2, :], axis=1, keepdims=True) + decv_r[2:3, 0:1]
        out_r[...] = jnp.broadcast_to(val, (_NB, 128))

    return pl.pallas_call(
        body,
        grid=(N // _NB,),
        in_specs=[pl.BlockSpec((_NB, 128), lambda i: (i, 0)),
                  _full((128, 128)), _full((8, 128))],
        out_specs=pl.BlockSpec((_NB, 128), lambda i: (i, 0)),
        out_shape=jax.ShapeDtypeStruct((N, 128), _f32),
    )(h, decW1, decv)


def kernel(x, edge_index, edge_attr, prb_data, unit_normal_vector, tags, params):
    p = params
    src = edge_index[0]
    dst = edge_index[1]
    keep = src != dst
    dst_m = jnp.where(keep, dst, N)
    src_m = jnp.where(keep, src, N)

    pad = E_PAD - E
    dst_p = jnp.pad(dst, (0, pad)).astype(jnp.int32)
    src_p = jnp.pad(src, (0, pad)).astype(jnp.int32)
    dstm_p = jnp.pad(dst_m, (0, pad), constant_values=N).astype(jnp.int32)
    srcm_p = jnp.pad(src_m, (0, pad), constant_values=N).astype(jnp.int32)

    def wpack(W1, b1):
        w = jnp.zeros((8, 128), _f32)
        return w.at[0:3, :].set(W1[256:259]).at[3, :].set(b1)

    wto = wpack(p['to_W1'], p['to_b1'])
    wfr = wpack(p['fr_W1'], p['fr_b1'])
    wnm = wpack(p['nm_W1'], p['nm_b1'])

    toA = p['to_W1'][:128]
    toB = p['to_W1'][128:256]
    frA, frB = p['fr_W1'][:128], p['fr_W1'][128:256]
    nmA, nmB = p['nm_W1'][:128], p['nm_W1'][128:256]
    wproj = jnp.concatenate([
        toA[:, :64], toA[:, 64:], toB[:, :64], toB[:, 64:],
        frA[:, :64], nmA[:, :64], frA[:, 64:], nmA[:, 64:],
        frB[:, :64], nmB[:, :64], frB[:, 64:], nmB[:, 64:],
    ], axis=1)

    encv = jnp.zeros((8, 128), _f32)
    encv = encv.at[0].set(p['enc_W1'][0]).at[1].set(p['enc_b1']).at[2].set(p['enc_b2'])
    decv = jnp.zeros((8, 128), _f32)
    decv = decv.at[0].set(p['dec_b1']).at[1].set(p['dec_W2'][:, 0]).at[2, 0].set(p['dec_b2'][0])

    al_misc = jnp.concatenate([p['al_W'][384:387, 0], p['al_b'],
                               jnp.zeros((124,), _f32)])
    vecs = jnp.stack([p['to_b2'], p['fr_b2'], p['nm_b2'], p['up_b1'],
                      p['up_b2'], p['un_b1'], p['un_b2'], p['ln_g'], p['ln_b'],
                      p['al_W'][0:128, 0], p['al_W'][128:256, 0],
                      p['al_W'][256:384, 0], al_misc,
                      jnp.zeros((128,), _f32), jnp.zeros((128,), _f32),
                      jnp.zeros((128,), _f32)])
    wext = jnp.concatenate([p['up_W1'][384:387], p['un_W1'][256:259],
                            p['un_W1'][259:261]], axis=0)

    W = (p['to_W2'], p['fr_W2'], p['nm_W2'],
         p['up_W1'][0:128], p['up_W1'][128:256], p['up_W1'][256:384], p['up_W2'],
         p['un_W1'][0:128], p['un_W1'][128:256], p['un_W2'],
         vecs, wext, wproj)

    c64, c128 = _tc_edge_consts(edge_attr, wto, wfr, wnm)
    degs = _sc_degrees(dstm_p, srcm_p)

    xp = jnp.zeros((N, 128), _f32).at[:, 0:1].set(x)
    h_init, toAt, toBt, fnAt, fnBt = _tc_encoder(xp, encv, p['enc_W2'], wproj)

    h = h_init
    for _ in range(ITERS):
        s_to = _sc_phi(toAt, toBt, c64, dst_p, src_p, dstm_p, 64, 128, 80)
        s_fn = _sc_phi(fnAt, fnBt, c128, src_p, dst_p, srcm_p, 128, 64, 160)
        h, toAt, toBt, fnAt, fnBt = _tc_update(
            h, h_init, s_to, s_fn, prb_data, unit_normal_vector, tags, degs, W)

    out = _tc_decoder(h, p['dec_W1'], decv)
    return out[:, 0:1]
